# baseline (speedup 1.0000x reference)
import functools

import jax
import jax.numpy as jnp
from jax import lax
from jax.experimental import pallas as pl
from jax.experimental.pallas import tpu as pltpu
from jax.experimental.pallas import tpu_sc as plsc

N = 10000
E = 160000
H = 128
STEPS = 50
CP = 64
EPS = 1e-5
SLOPE = 0.01
E_F = float(E)

RB = 3200
NBLK = E // RB
K = 128
NCHUNK = E // K
K3 = 80
NCHUNK3 = E // K3
NC, NS = 2, 16
NW = NC * NS
NPAD = 10240
ROWS_PER_TILE = NPAD // NS

_f32 = jnp.float32


def _leaky(x):
    return jnp.maximum(x, x * SLOPE)


def _sc_p1_body(a_hbm, c_hbm, g_hbm, src_hbm, dst_hbm, y_hbm, st_hbm,
                idx_s0, idx_d0, a_v0, c_v0, g_v0,
                idx_s1, idx_d1, a_v1, c_v1, g_v1,
                st_v, semg0, semg1, semy0, semy1):
    cid = lax.axis_index("c")
    sid = lax.axis_index("s")
    w = cid * NS + sid
    nchunks = (NCHUNK - w + NW - 1) // NW
    bufs = ((idx_s0, idx_d0, a_v0, c_v0, g_v0, semg0, semy0),
            (idx_s1, idx_d1, a_v1, c_v1, g_v1, semg1, semy1))
    zero = jnp.zeros((16,), _f32)
    init = (zero,) * 16
    for k in range(16):
        st_v[pl.ds(k * 16, 16)] = zero

    def cbase(k):
        return (w + k * NW) * K

    def fire(k, buf):
        idx_s, idx_d, a_v, c_v, g_v, semg, _ = buf
        base = cbase(k)
        pltpu.sync_copy(src_hbm.at[pl.ds(base, K)], idx_s)
        pltpu.sync_copy(dst_hbm.at[pl.ds(base, K)], idx_d)
        pltpu.async_copy(a_hbm.at[idx_s], a_v, semg)
        pltpu.async_copy(c_hbm.at[idx_d], c_v, semg)
        pltpu.async_copy(g_hbm.at[pl.ds(base, K)], g_v, semg)

    def process(k, b):
        idx_s, idx_d, a_v, c_v, g_v, semg, semy = bufs[b]
        idx_so, _, a_vo, _, _, _, semyo = bufs[1 - b]

        @pl.when(k >= 1)
        def _():
            pltpu.make_async_copy(y_hbm.at[pl.ds(0, K)], a_vo, semyo).wait()

        @pl.when(k + 1 < nchunks)
        def _():
            fire(k + 1, bufs[1 - b])

        base = cbase(k)
        pltpu.make_async_copy(a_hbm.at[idx_s], a_v, semg).wait()
        pltpu.make_async_copy(c_hbm.at[idx_d], c_v, semg).wait()
        pltpu.make_async_copy(g_hbm.at[pl.ds(base, K)], g_v, semg).wait()

        def row_body(i, st_in):
            out = list(st_in)
            for kk in range(8):
                sl = pl.ds(kk * 16, 16)
                p = a_v[i, sl] + c_v[i, sl] + g_v[i, sl]
                yv = jnp.maximum(p, p * SLOPE)
                a_v[i, sl] = yv
                out[kk] = out[kk] + yv
                out[8 + kk] = out[8 + kk] + yv * yv
            return tuple(out)

        st = lax.fori_loop(0, K, row_body, init)
        for kk in range(16):
            sl = pl.ds(kk * 16, 16)
            st_v[sl] = st_v[sl] + st[kk]
        pltpu.async_copy(a_v, y_hbm.at[pl.ds(base, K)], semy)

    fire(0, bufs[0])

    def pair_body(p, carry):
        process(2 * p, 0)

        @pl.when(2 * p + 1 < nchunks)
        def _():
            process(2 * p + 1, 1)

        return carry

    lax.fori_loop(0, (nchunks + 1) // 2, pair_body, 0)
    last = (nchunks - 1) % 2

    @pl.when(last == 0)
    def _():
        pltpu.make_async_copy(y_hbm.at[pl.ds(0, K)], a_v0, semy0).wait()

    @pl.when(last == 1)
    def _():
        pltpu.make_async_copy(y_hbm.at[pl.ds(0, K)], a_v1, semy1).wait()

    pltpu.sync_copy(st_v, st_hbm.at[pl.ds(w * 2 * H, 2 * H)])


def _sc_p1(A, C, G, src, dst):
    mesh = plsc.VectorSubcoreMesh(core_axis_name="c", subcore_axis_name="s")
    buf_set = [
        pltpu.VMEM((K,), jnp.int32),
        pltpu.VMEM((K,), jnp.int32),
        pltpu.VMEM((K, H), _f32),
        pltpu.VMEM((K, H), _f32),
        pltpu.VMEM((K, H), _f32),
    ]
    f = pl.kernel(
        _sc_p1_body,
        out_type=(jax.ShapeDtypeStruct((E, H), _f32),
                  jax.ShapeDtypeStruct((NW * 2 * H,), _f32)),
        mesh=mesh,
        scratch_types=buf_set + buf_set + [
            pltpu.VMEM((2 * H,), _f32),
            pltpu.SemaphoreType.DMA,
            pltpu.SemaphoreType.DMA,
            pltpu.SemaphoreType.DMA,
            pltpu.SemaphoreType.DMA,
        ],
    )
    return f(A, C, G, src, dst)


def _sc_p3_body(cond_hbm, d_hbm, src_hbm, dst_hbm, out_hbm,
                idx_s0, idx_d0, m_v0, d_v0,
                idx_s1, idx_d1, m_v1, d_v1,
                acc, semg0, semg1, semsc0, semsc1):
    cid = lax.axis_index("c")
    sid = lax.axis_index("s")
    w = cid * NS + sid
    bufs = ((idx_s0, idx_d0, m_v0, d_v0, semg0, semsc0),
            (idx_s1, idx_d1, m_v1, d_v1, semg1, semsc1))

    def zrow(i, _):
        for k in range(8):
            m_v0[i, pl.ds(k * 16, 16)] = jnp.zeros((16,), _f32)
        return 0

    lax.fori_loop(0, K3, zrow, 0)
    base_r = sid * ROWS_PER_TILE
    for t in range(ROWS_PER_TILE // K3):
        pltpu.sync_copy(m_v0, acc.at[pl.ds(base_r + t * K3, K3)])
    plsc.subcore_barrier()

    nchunks = (NCHUNK3 - w + NW - 1) // NW

    def cbase(k):
        return (w + k * NW) * K3

    def fire(k, buf):
        idx_s, idx_d, m_v, d_v, semg, _ = buf
        base = cbase(k)
        pltpu.sync_copy(src_hbm.at[pl.ds(base, K3)], idx_s)
        pltpu.sync_copy(dst_hbm.at[pl.ds(base, K3)], idx_d)
        pltpu.async_copy(d_hbm.at[idx_s], d_v, semg)
        pltpu.async_copy(cond_hbm.at[pl.ds(base, K3)], m_v, semg)

    def process(k, b):
        idx_s, idx_d, m_v, d_v, semg, semsc = bufs[b]
        _, idx_do, m_vo, _, _, semsco = bufs[1 - b]

        @pl.when(k >= 1)
        def _():
            pltpu.make_async_copy(m_vo, acc.at[idx_do], semsco).wait()

        @pl.when(k + 1 < nchunks)
        def _():
            fire(k + 1, bufs[1 - b])

        base = cbase(k)
        pltpu.make_async_copy(d_hbm.at[idx_s], d_v, semg).wait()
        pltpu.make_async_copy(cond_hbm.at[pl.ds(base, K3)], m_v, semg).wait()

        def row_body(i, _):
            for kk in range(8):
                sl = pl.ds(kk * 16, 16)
                m_v[i, sl] = m_v[i, sl] * d_v[i, sl]
            return 0

        lax.fori_loop(0, K3, row_body, 0)
        pltpu.async_copy(m_v, acc.at[idx_d], semsc, add=True)

    fire(0, bufs[0])

    def pair_body(p, carry):
        process(2 * p, 0)

        @pl.when(2 * p + 1 < nchunks)
        def _():
            process(2 * p + 1, 1)

        return carry

    lax.fori_loop(0, (nchunks + 1) // 2, pair_body, 0)
    last = (nchunks - 1) % 2

    @pl.when(last == 0)
    def _():
        pltpu.make_async_copy(m_v0, acc.at[idx_d0], semsc0).wait()

    @pl.when(last == 1)
    def _():
        pltpu.make_async_copy(m_v1, acc.at[idx_d1], semsc1).wait()

    plsc.subcore_barrier()

    out_base = cid * NPAD + base_r
    for t in range(ROWS_PER_TILE // K3):
        pltpu.sync_copy(acc.at[pl.ds(base_r + t * K3, K3)], m_v0)
        pltpu.sync_copy(m_v0, out_hbm.at[pl.ds(out_base + t * K3, K3)])


def _sc_p3(cond, D, src, dst):
    mesh = plsc.VectorSubcoreMesh(core_axis_name="c", subcore_axis_name="s")
    buf_set = [
        pltpu.VMEM((K3,), jnp.int32),
        pltpu.VMEM((K3,), jnp.int32),
        pltpu.VMEM((K3, H), _f32),
        pltpu.VMEM((K3, H), _f32),
    ]
    f = pl.kernel(
        _sc_p3_body,
        out_type=jax.ShapeDtypeStruct((NC * NPAD, H), _f32),
        mesh=mesh,
        scratch_types=buf_set + buf_set + [
            pltpu.VMEM_SHARED((NPAD, H), _f32),
            pltpu.SemaphoreType.DMA,
            pltpu.SemaphoreType.DMA,
            pltpu.SemaphoreType.DMA,
            pltpu.SemaphoreType.DMA,
        ],
    )
    return f(cond, D, src, dst)


def _tc_pre_body(ea_ref, ea4_ref, sel_ref, wfbp_ref, bfb_ref, w1m_ref, b1_ref,
                 g0_ref, g1_ref, g2_ref, coef_ref):
    ea = ea_ref[...]
    d = jnp.dot(ea, sel_ref[...], preferred_element_type=_f32,
                precision=lax.Precision.HIGHEST)
    dexp = jnp.exp(-(d * d) * STEPS)
    g = jnp.dot(dexp, wfbp_ref[...], preferred_element_type=_f32) + bfb_ref[...]
    for l, gref in enumerate((g0_ref, g1_ref, g2_ref)):
        gref[...] = jnp.dot(g, w1m_ref[pl.ds(l * H, H), :],
                            preferred_element_type=_f32) + b1_ref[l:l + 1, :]
    coef_ref[...] = jnp.cos(1.5707963267948966 * ea4_ref[...])


def _tc_pre(ea_aug, ea4d, sel, wfb_pad, bfb, w1m_all, b1_all):
    out_shape = (jax.ShapeDtypeStruct((E, H), _f32),) * 3 + (
        jax.ShapeDtypeStruct((NBLK, RB // H, H), _f32),)
    full = lambda shp: pl.BlockSpec(shp, lambda i: (0, 0))
    return pl.pallas_call(
        _tc_pre_body,
        grid=(NBLK,),
        in_specs=[pl.BlockSpec((RB, 8), lambda i: (i, 0)),
                  pl.BlockSpec((1, RB // H, H), lambda i: (i, 0, 0)),
                  full((8, 7 * CP)), full((7 * CP, H)), full((1, H)),
                  full((3 * H, H)), full((3, H))],
        out_specs=[pl.BlockSpec((RB, H), lambda i: (i, 0))] * 3 +
                  [pl.BlockSpec((1, RB // H, H), lambda i: (i, 0, 0))],
        out_shape=out_shape,
    )(ea_aug, ea4d, sel, wfb_pad, bfb, w1m_all, b1_all)


def _tc_init_body(x_ref, w_ref, b_ref, h_ref):
    h_ref[...] = jnp.dot(x_ref[...], w_ref[...],
                         preferred_element_type=_f32) + b_ref[...]


def _tc_init(x, w, b):
    return pl.pallas_call(
        _tc_init_body,
        out_shape=jax.ShapeDtypeStruct((N, H), _f32),
    )(x, w, b)


def _tc_nodepre_body(h_ref, wa_ref, wc_ref, wd_ref, bd_ref, a_ref, c_ref, d_ref):
    h = h_ref[...]
    a_ref[...] = jnp.dot(h, wa_ref[...], preferred_element_type=_f32)
    c_ref[...] = jnp.dot(h, wc_ref[...], preferred_element_type=_f32)
    d_ref[...] = jnp.dot(h, wd_ref[...], preferred_element_type=_f32) + bd_ref[...]


def _tc_nodepre(h, wa, wc, wd, bd):
    return pl.pallas_call(
        _tc_nodepre_body,
        out_shape=(jax.ShapeDtypeStruct((N, H), _f32),) * 3,
    )(h, wa, wc, wd, bd)


def _bn_scale_shift(st_row_sum, st_row_sq, g, b):
    m = st_row_sum / E_F
    var = st_row_sq / E_F - m * m
    sc = g * lax.rsqrt(var + EPS)
    sh = b - m * sc
    return sc, sh


def _tc_p2_body(st_ref, bng_ref, bnb_ref, w2_ref, b2_ref, y1_ref,
                y2_ref, st2_ref):
    i = pl.program_id(0)
    sc, sh = _bn_scale_shift(st_ref[0:1, :], st_ref[1:2, :],
                             bng_ref[...], bnb_ref[...])
    y1n = y1_ref[...] * sc + sh
    u = jnp.dot(y1n, w2_ref[...], preferred_element_type=_f32) + b2_ref[...]
    y2 = jnp.maximum(u, u * SLOPE)
    y2_ref[...] = y2

    @pl.when(i == 0)
    def _():
        st2_ref[...] = jnp.zeros_like(st2_ref)

    st2_ref[0:1, :] += jnp.sum(y2, axis=0, keepdims=True)
    st2_ref[1:2, :] += jnp.sum(y2 * y2, axis=0, keepdims=True)


def _tc_p2(st1, bng, bnb, w2, b2, y1):
    full = lambda shp: pl.BlockSpec(shp, lambda i: (0, 0))
    return pl.pallas_call(
        _tc_p2_body,
        grid=(NBLK,),
        in_specs=[full((2, H)), full((1, H)), full((1, H)),
                  full((H, H)), full((1, H)),
                  pl.BlockSpec((RB, H), lambda i: (i, 0))],
        out_specs=[pl.BlockSpec((RB, H), lambda i: (i, 0)),
                   pl.BlockSpec((8, H), lambda i: (0, 0))],
        out_shape=(jax.ShapeDtypeStruct((E, H), _f32),
                   jax.ShapeDtypeStruct((8, H), _f32)),
    )(st1, bng, bnb, w2, b2, y1)


def _tc_p3a_body(st2_ref, bng_ref, bnb_ref, w3_ref, b3_ref, y2_ref, coef_ref,
                 cond_ref):
    sc, sh = _bn_scale_shift(st2_ref[0:1, :], st2_ref[1:2, :],
                             bng_ref[...], bnb_ref[...])
    y2n = y2_ref[...] * sc + sh
    fe = jnp.dot(y2n, w3_ref[...], preferred_element_type=_f32) + b3_ref[...]
    cond_ref[...] = fe * coef_ref[...]


def _tc_p3a(st2, bng, bnb, w3, b3, y2, coef):
    full = lambda shp: pl.BlockSpec(shp, lambda i: (0, 0))
    return pl.pallas_call(
        _tc_p3a_body,
        grid=(NBLK,),
        in_specs=[full((8, H)), full((1, H)), full((1, H)),
                  full((H, H)), full((1, H)),
                  pl.BlockSpec((RB, H), lambda i: (i, 0)),
                  pl.BlockSpec((RB, 1), lambda i: (i, 0))],
        out_specs=pl.BlockSpec((RB, H), lambda i: (i, 0)),
        out_shape=jax.ShapeDtypeStruct((E, H), _f32),
    )(st2, bng, bnb, w3, b3, y2, coef)


def _tc_node_body(p0_ref, p1_ref, d_ref, h_ref, v_ref, w0_ref, b0_ref,
                  g_ref, bb_ref, w2_ref, b2_ref, ho_ref):
    z = v_ref[...] * d_ref[...] + p0_ref[...] + p1_ref[...]
    z1 = jnp.dot(z, w0_ref[...], preferred_element_type=_f32) + b0_ref[...]
    z1 = _leaky(z1)
    m = jnp.mean(z1, axis=0, keepdims=True)
    cz = z1 - m
    var = jnp.mean(cz * cz, axis=0, keepdims=True)
    z1n = cz * lax.rsqrt(var + EPS) * g_ref[...] + bb_ref[...]
    z2 = jnp.dot(z1n, w2_ref[...], preferred_element_type=_f32) + b2_ref[...]
    ho_ref[...] = z2 + h_ref[...]


def _tc_node(p0, p1, D, h, v, w0, b0, g, bb, w2, b2):
    return pl.pallas_call(
        _tc_node_body,
        out_shape=jax.ShapeDtypeStruct((N, H), _f32),
    )(p0, p1, D, h, v, w0, b0, g, bb, w2, b2)


def _tc_head_body(h_ref, w0_ref, b0_ref, g_ref, bb_ref, w1_ref, b1_ref, y_ref,
                  loss_ref, pred_ref):
    t = jnp.dot(h_ref[...], w0_ref[...], preferred_element_type=_f32) + b0_ref[...]
    t = _leaky(t)
    m = jnp.mean(t, axis=0, keepdims=True)
    ct = t - m
    var = jnp.mean(ct * ct, axis=0, keepdims=True)
    tn = ct * lax.rsqrt(var + EPS) * g_ref[...] + bb_ref[...]
    pred = jnp.dot(tn, w1_ref[...], preferred_element_type=_f32) + b1_ref[...]
    pred_ref[...] = pred
    r = pred - y_ref[...]
    loss_ref[...] = jnp.mean(r * r).reshape(1, 1)


def _tc_head(h, w0, b0, g, bb, w1, b1, y):
    return pl.pallas_call(
        _tc_head_body,
        out_shape=(jax.ShapeDtypeStruct((1, 1), _f32),
                   jax.ShapeDtypeStruct((N, 1), _f32)),
    )(h, w0, b0, g, bb, w1, b1, y)


def _row(v):
    return v.reshape(1, -1).astype(_f32)


def kernel(x, edge_attr, edge_index, y, params):
    x = x.astype(_f32)
    edge_attr = edge_attr.astype(_f32)
    y = y.astype(_f32)
    src = edge_index[0].astype(jnp.int32)
    dst = edge_index[1].astype(jnp.int32)

    p = params
    wfb = p["f_b"]["W"].astype(_f32)
    wfb_pad = jnp.zeros((7, CP, H), _f32).at[:, :STEPS, :].set(
        wfb.reshape(7, STEPS, H)).reshape(7 * CP, H)
    w1m_all = jnp.concatenate(
        [p["layers"][l]["f_e"][0]["W"][H:2 * H, :] for l in range(3)], axis=0)
    b1_all = jnp.stack([p["layers"][l]["f_e"][0]["b"] for l in range(3)], axis=0)

    ea_aug = jnp.concatenate([edge_attr, jnp.ones((E, 1), _f32)], axis=1)
    centers = (jnp.arange(CP, dtype=_f32) / (STEPS - 1.0))
    sel = jnp.zeros((8, 7 * CP), _f32)
    for a in range(7):
        sel = sel.at[a, a * CP:(a + 1) * CP].set(1.0)
        sel = sel.at[7, a * CP:(a + 1) * CP].set(-centers)
    ea4d = edge_attr[:, 3].reshape(NBLK, RB // H, H)
    g0, g1, g2, coef2 = _tc_pre(ea_aug, ea4d, sel, wfb_pad, _row(p["f_b"]["b"]),
                                w1m_all.astype(_f32), b1_all.astype(_f32))
    coef = coef2.reshape(E, 1)
    gmids = (g0, g1, g2)

    h = _tc_init(x, p["f_x"]["W"].astype(_f32), _row(p["f_x"]["b"]))

    for l in range(3):
        lp = p["layers"][l]
        w1 = lp["f_e"][0]["W"].astype(_f32)
        A, C, D = _tc_nodepre(h, w1[:H, :], w1[2 * H:, :],
                              lp["f_d"]["W"].astype(_f32), _row(lp["f_d"]["b"]))
        y1, st1p = _sc_p1(A, C, gmids[l], src, dst)
        st1 = st1p.reshape(NW, 2, H).sum(axis=0)
        y2, st2 = _tc_p2(st1, _row(lp["f_e"][1]["g"]), _row(lp["f_e"][1]["b"]),
                         lp["f_e"][2]["W"].astype(_f32), _row(lp["f_e"][2]["b"]),
                         y1)
        cond = _tc_p3a(st2, _row(lp["f_e"][3]["g"]), _row(lp["f_e"][3]["b"]),
                       lp["f_e"][4]["W"].astype(_f32), _row(lp["f_e"][4]["b"]),
                       y2, coef)
        parts = _sc_p3(cond, D, src, dst)
        h = _tc_node(parts[:N], parts[NPAD:NPAD + N], D, h, lp["v"].astype(_f32),
                     lp["f_n"][0]["W"].astype(_f32), _row(lp["f_n"][0]["b"]),
                     _row(lp["f_n"][1]["g"]), _row(lp["f_n"][1]["b"]),
                     lp["f_n"][2]["W"].astype(_f32), _row(lp["f_n"][2]["b"]))

    ft = p["f_target"]
    loss, pred = _tc_head(h, ft[0]["W"].astype(_f32), _row(ft[0]["b"]),
                          _row(ft[1]["g"]), _row(ft[1]["b"]),
                          ft[2]["W"].astype(_f32), _row(ft[2]["b"]), y)
    return loss[0, 0], pred


# speedup vs baseline: 1.0969x; 1.0969x over previous
import functools

import jax
import jax.numpy as jnp
from jax import lax
from jax.experimental import pallas as pl
from jax.experimental.pallas import tpu as pltpu
from jax.experimental.pallas import tpu_sc as plsc

N = 10000
E = 160000
H = 128
STEPS = 50
CP = 64
EPS = 1e-5
SLOPE = 0.01
E_F = float(E)

RB = 3200
NBLK = E // RB
K = 128
NCHUNK = E // K
K3 = 80
NCHUNK3 = E // K3
NC, NS = 2, 16
NW = NC * NS
NPAD = 10240
ROWS_PER_TILE = NPAD // NS

_f32 = jnp.float32


def _leaky(x):
    return jnp.maximum(x, x * SLOPE)


def _sc_p1_body(a_hbm, c_hbm, g_hbm, src_hbm, dst_hbm, y_hbm, st_hbm,
                idx_s0, idx_d0, a_v0, c_v0, g_v0,
                idx_s1, idx_d1, a_v1, c_v1, g_v1,
                st_v, semg0, semg1, semy0, semy1):
    cid = lax.axis_index("c")
    sid = lax.axis_index("s")
    w = cid * NS + sid
    nchunks = (NCHUNK - w + NW - 1) // NW
    bufs = ((idx_s0, idx_d0, a_v0, c_v0, g_v0, semg0, semy0),
            (idx_s1, idx_d1, a_v1, c_v1, g_v1, semg1, semy1))
    zero = jnp.zeros((16,), _f32)
    init = (zero,) * 16
    for k in range(16):
        st_v[pl.ds(k * 16, 16)] = zero

    def cbase(k):
        return (w + k * NW) * K

    def fire(k, buf):
        idx_s, idx_d, a_v, c_v, g_v, semg, _ = buf
        base = cbase(k)
        pltpu.sync_copy(src_hbm.at[pl.ds(base, K)], idx_s)
        pltpu.sync_copy(dst_hbm.at[pl.ds(base, K)], idx_d)
        pltpu.async_copy(a_hbm.at[idx_s], a_v, semg)
        pltpu.async_copy(c_hbm.at[idx_d], c_v, semg)
        pltpu.async_copy(g_hbm.at[pl.ds(base, K)], g_v, semg)

    def process(k, b):
        idx_s, idx_d, a_v, c_v, g_v, semg, semy = bufs[b]
        idx_so, _, a_vo, _, _, _, semyo = bufs[1 - b]

        @pl.when(k >= 1)
        def _():
            pltpu.make_async_copy(y_hbm.at[pl.ds(0, K)], a_vo, semyo).wait()

        @pl.when(k + 1 < nchunks)
        def _():
            fire(k + 1, bufs[1 - b])

        base = cbase(k)
        pltpu.make_async_copy(a_hbm.at[idx_s], a_v, semg).wait()
        pltpu.make_async_copy(c_hbm.at[idx_d], c_v, semg).wait()
        pltpu.make_async_copy(g_hbm.at[pl.ds(base, K)], g_v, semg).wait()

        def row_body(i, st_in):
            out = list(st_in)
            for kk in range(8):
                sl = pl.ds(kk * 16, 16)
                p = a_v[i, sl] + c_v[i, sl] + g_v[i, sl]
                yv = jnp.maximum(p, p * SLOPE)
                a_v[i, sl] = yv
                out[kk] = out[kk] + yv
                out[8 + kk] = out[8 + kk] + yv * yv
            return tuple(out)

        st = lax.fori_loop(0, K, row_body, init)
        for kk in range(16):
            sl = pl.ds(kk * 16, 16)
            st_v[sl] = st_v[sl] + st[kk]
        pltpu.async_copy(a_v, y_hbm.at[pl.ds(base, K)], semy)

    fire(0, bufs[0])

    def pair_body(p, carry):
        process(2 * p, 0)

        @pl.when(2 * p + 1 < nchunks)
        def _():
            process(2 * p + 1, 1)

        return carry

    lax.fori_loop(0, (nchunks + 1) // 2, pair_body, 0)
    last = (nchunks - 1) % 2

    @pl.when(last == 0)
    def _():
        pltpu.make_async_copy(y_hbm.at[pl.ds(0, K)], a_v0, semy0).wait()

    @pl.when(last == 1)
    def _():
        pltpu.make_async_copy(y_hbm.at[pl.ds(0, K)], a_v1, semy1).wait()

    pltpu.sync_copy(st_v, st_hbm.at[pl.ds(w * 2 * H, 2 * H)])


def _sc_p1(A, C, G, src, dst):
    mesh = plsc.VectorSubcoreMesh(core_axis_name="c", subcore_axis_name="s")
    buf_set = [
        pltpu.VMEM((K,), jnp.int32),
        pltpu.VMEM((K,), jnp.int32),
        pltpu.VMEM((K, H), _f32),
        pltpu.VMEM((K, H), _f32),
        pltpu.VMEM((K, H), _f32),
    ]
    f = pl.kernel(
        _sc_p1_body,
        out_type=(jax.ShapeDtypeStruct((E, H), _f32),
                  jax.ShapeDtypeStruct((NW * 2 * H,), _f32)),
        mesh=mesh,
        scratch_types=buf_set + buf_set + [
            pltpu.VMEM((2 * H,), _f32),
            pltpu.SemaphoreType.DMA,
            pltpu.SemaphoreType.DMA,
            pltpu.SemaphoreType.DMA,
            pltpu.SemaphoreType.DMA,
        ],
    )
    return f(A, C, G, src, dst)


def _sc_p3_body(cond_hbm, d_hbm, src_hbm, dst_hbm, out_hbm,
                idx_s0, idx_d0, m_v0, d_v0,
                idx_s1, idx_d1, m_v1, d_v1,
                acc, semg0, semg1, semsc0, semsc1):
    cid = lax.axis_index("c")
    sid = lax.axis_index("s")
    w = cid * NS + sid
    bufs = ((idx_s0, idx_d0, m_v0, d_v0, semg0, semsc0),
            (idx_s1, idx_d1, m_v1, d_v1, semg1, semsc1))

    def zrow(i, _):
        for k in range(8):
            m_v0[i, pl.ds(k * 16, 16)] = jnp.zeros((16,), _f32)
        return 0

    lax.fori_loop(0, K3, zrow, 0)
    base_r = sid * ROWS_PER_TILE
    for t in range(ROWS_PER_TILE // K3):
        pltpu.sync_copy(m_v0, acc.at[pl.ds(base_r + t * K3, K3)])
    plsc.subcore_barrier()

    nchunks = (NCHUNK3 - w + NW - 1) // NW

    def cbase(k):
        return (w + k * NW) * K3

    def fire(k, buf):
        idx_s, idx_d, m_v, d_v, semg, _ = buf
        base = cbase(k)
        pltpu.sync_copy(src_hbm.at[pl.ds(base, K3)], idx_s)
        pltpu.sync_copy(dst_hbm.at[pl.ds(base, K3)], idx_d)
        pltpu.async_copy(d_hbm.at[idx_s], d_v, semg)
        pltpu.async_copy(cond_hbm.at[pl.ds(base, K3)], m_v, semg)

    def process(k, b):
        idx_s, idx_d, m_v, d_v, semg, semsc = bufs[b]
        _, idx_do, m_vo, _, _, semsco = bufs[1 - b]

        @pl.when(k >= 1)
        def _():
            pltpu.make_async_copy(m_vo, acc.at[idx_do], semsco).wait()

        @pl.when(k + 1 < nchunks)
        def _():
            fire(k + 1, bufs[1 - b])

        base = cbase(k)
        pltpu.make_async_copy(d_hbm.at[idx_s], d_v, semg).wait()
        pltpu.make_async_copy(cond_hbm.at[pl.ds(base, K3)], m_v, semg).wait()

        def row_body(i, _):
            for kk in range(8):
                sl = pl.ds(kk * 16, 16)
                m_v[i, sl] = m_v[i, sl] * d_v[i, sl]
            return 0

        lax.fori_loop(0, K3, row_body, 0)
        pltpu.async_copy(m_v, acc.at[idx_d], semsc, add=True)

    fire(0, bufs[0])

    def pair_body(p, carry):
        process(2 * p, 0)

        @pl.when(2 * p + 1 < nchunks)
        def _():
            process(2 * p + 1, 1)

        return carry

    lax.fori_loop(0, (nchunks + 1) // 2, pair_body, 0)
    last = (nchunks - 1) % 2

    @pl.when(last == 0)
    def _():
        pltpu.make_async_copy(m_v0, acc.at[idx_d0], semsc0).wait()

    @pl.when(last == 1)
    def _():
        pltpu.make_async_copy(m_v1, acc.at[idx_d1], semsc1).wait()

    plsc.subcore_barrier()

    out_base = cid * NPAD + base_r
    for t in range(ROWS_PER_TILE // K3):
        pltpu.sync_copy(acc.at[pl.ds(base_r + t * K3, K3)], m_v0)
        pltpu.sync_copy(m_v0, out_hbm.at[pl.ds(out_base + t * K3, K3)])


def _sc_p3(cond, D, src, dst):
    mesh = plsc.VectorSubcoreMesh(core_axis_name="c", subcore_axis_name="s")
    buf_set = [
        pltpu.VMEM((K3,), jnp.int32),
        pltpu.VMEM((K3,), jnp.int32),
        pltpu.VMEM((K3, H), _f32),
        pltpu.VMEM((K3, H), _f32),
    ]
    f = pl.kernel(
        _sc_p3_body,
        out_type=jax.ShapeDtypeStruct((NC * NPAD, H), _f32),
        mesh=mesh,
        scratch_types=buf_set + buf_set + [
            pltpu.VMEM_SHARED((NPAD, H), _f32),
            pltpu.SemaphoreType.DMA,
            pltpu.SemaphoreType.DMA,
            pltpu.SemaphoreType.DMA,
            pltpu.SemaphoreType.DMA,
        ],
    )
    return f(cond, D, src, dst)


def _tc_pre_body(ea_ref, ea4_ref, sel_ref, wfbp_ref, bfb_ref, w1m_ref, b1_ref,
                 g0_ref, g1_ref, g2_ref, coef_ref):
    ea = ea_ref[...]
    centers = lax.broadcasted_iota(jnp.int32, (1, CP), 1).astype(_f32) / (STEPS - 1.0)
    g = jnp.zeros((RB, H), _f32) + bfb_ref[...]
    for a in range(7):
        col = ea[:, a:a + 1]
        dexp = jnp.exp(-((col - centers) ** 2) * STEPS)
        g = g + jnp.dot(dexp, wfbp_ref[pl.ds(a * CP, CP), :],
                        preferred_element_type=_f32)
    for l, gref in enumerate((g0_ref, g1_ref, g2_ref)):
        gref[...] = jnp.dot(g, w1m_ref[pl.ds(l * H, H), :],
                            preferred_element_type=_f32) + b1_ref[l:l + 1, :]
    coef_ref[...] = jnp.cos(1.5707963267948966 * ea4_ref[...])


def _tc_pre(ea_aug, ea4d, sel, wfb_pad, bfb, w1m_all, b1_all):
    out_shape = (jax.ShapeDtypeStruct((E, H), _f32),) * 3 + (
        jax.ShapeDtypeStruct((NBLK, RB // H, H), _f32),)
    full = lambda shp: pl.BlockSpec(shp, lambda i: (0, 0))
    return pl.pallas_call(
        _tc_pre_body,
        grid=(NBLK,),
        in_specs=[pl.BlockSpec((RB, 8), lambda i: (i, 0)),
                  pl.BlockSpec((1, RB // H, H), lambda i: (i, 0, 0)),
                  full((8, 7 * CP)), full((7 * CP, H)), full((1, H)),
                  full((3 * H, H)), full((3, H))],
        out_specs=[pl.BlockSpec((RB, H), lambda i: (i, 0))] * 3 +
                  [pl.BlockSpec((1, RB // H, H), lambda i: (i, 0, 0))],
        out_shape=out_shape,
    )(ea_aug, ea4d, sel, wfb_pad, bfb, w1m_all, b1_all)


def _tc_init_body(x_ref, w_ref, b_ref, h_ref):
    h_ref[...] = jnp.dot(x_ref[...], w_ref[...],
                         preferred_element_type=_f32) + b_ref[...]


def _tc_init(x, w, b):
    return pl.pallas_call(
        _tc_init_body,
        out_shape=jax.ShapeDtypeStruct((N, H), _f32),
    )(x, w, b)


def _tc_nodepre_body(h_ref, wa_ref, wc_ref, wd_ref, bd_ref, a_ref, c_ref, d_ref):
    h = h_ref[...]
    a_ref[...] = jnp.dot(h, wa_ref[...], preferred_element_type=_f32)
    c_ref[...] = jnp.dot(h, wc_ref[...], preferred_element_type=_f32)
    d_ref[...] = jnp.dot(h, wd_ref[...], preferred_element_type=_f32) + bd_ref[...]


def _tc_nodepre(h, wa, wc, wd, bd):
    return pl.pallas_call(
        _tc_nodepre_body,
        out_shape=(jax.ShapeDtypeStruct((N, H), _f32),) * 3,
    )(h, wa, wc, wd, bd)


def _bn_scale_shift(st_row_sum, st_row_sq, g, b):
    m = st_row_sum / E_F
    var = st_row_sq / E_F - m * m
    sc = g * lax.rsqrt(var + EPS)
    sh = b - m * sc
    return sc, sh


def _tc_p2_body(st_ref, bng_ref, bnb_ref, w2_ref, b2_ref, y1_ref,
                y2_ref, st2_ref):
    i = pl.program_id(0)
    sc, sh = _bn_scale_shift(st_ref[0:1, :], st_ref[1:2, :],
                             bng_ref[...], bnb_ref[...])
    y1n = y1_ref[...] * sc + sh
    u = jnp.dot(y1n, w2_ref[...], preferred_element_type=_f32) + b2_ref[...]
    y2 = jnp.maximum(u, u * SLOPE)
    y2_ref[...] = y2

    @pl.when(i == 0)
    def _():
        st2_ref[...] = jnp.zeros_like(st2_ref)

    st2_ref[0:1, :] += jnp.sum(y2, axis=0, keepdims=True)
    st2_ref[1:2, :] += jnp.sum(y2 * y2, axis=0, keepdims=True)


def _tc_p2(st1, bng, bnb, w2, b2, y1):
    full = lambda shp: pl.BlockSpec(shp, lambda i: (0, 0))
    return pl.pallas_call(
        _tc_p2_body,
        grid=(NBLK,),
        in_specs=[full((2, H)), full((1, H)), full((1, H)),
                  full((H, H)), full((1, H)),
                  pl.BlockSpec((RB, H), lambda i: (i, 0))],
        out_specs=[pl.BlockSpec((RB, H), lambda i: (i, 0)),
                   pl.BlockSpec((8, H), lambda i: (0, 0))],
        out_shape=(jax.ShapeDtypeStruct((E, H), _f32),
                   jax.ShapeDtypeStruct((8, H), _f32)),
    )(st1, bng, bnb, w2, b2, y1)


def _tc_p3a_body(st2_ref, bng_ref, bnb_ref, w3_ref, b3_ref, y2_ref, coef_ref,
                 cond_ref):
    sc, sh = _bn_scale_shift(st2_ref[0:1, :], st2_ref[1:2, :],
                             bng_ref[...], bnb_ref[...])
    y2n = y2_ref[...] * sc + sh
    fe = jnp.dot(y2n, w3_ref[...], preferred_element_type=_f32) + b3_ref[...]
    cond_ref[...] = fe * coef_ref[...]


def _tc_p3a(st2, bng, bnb, w3, b3, y2, coef):
    full = lambda shp: pl.BlockSpec(shp, lambda i: (0, 0))
    return pl.pallas_call(
        _tc_p3a_body,
        grid=(NBLK,),
        in_specs=[full((8, H)), full((1, H)), full((1, H)),
                  full((H, H)), full((1, H)),
                  pl.BlockSpec((RB, H), lambda i: (i, 0)),
                  pl.BlockSpec((RB, 1), lambda i: (i, 0))],
        out_specs=pl.BlockSpec((RB, H), lambda i: (i, 0)),
        out_shape=jax.ShapeDtypeStruct((E, H), _f32),
    )(st2, bng, bnb, w3, b3, y2, coef)


def _tc_node_body(p0_ref, p1_ref, d_ref, h_ref, v_ref, w0_ref, b0_ref,
                  g_ref, bb_ref, w2_ref, b2_ref, ho_ref):
    z = v_ref[...] * d_ref[...] + p0_ref[...] + p1_ref[...]
    z1 = jnp.dot(z, w0_ref[...], preferred_element_type=_f32) + b0_ref[...]
    z1 = _leaky(z1)
    m = jnp.mean(z1, axis=0, keepdims=True)
    cz = z1 - m
    var = jnp.mean(cz * cz, axis=0, keepdims=True)
    z1n = cz * lax.rsqrt(var + EPS) * g_ref[...] + bb_ref[...]
    z2 = jnp.dot(z1n, w2_ref[...], preferred_element_type=_f32) + b2_ref[...]
    ho_ref[...] = z2 + h_ref[...]


def _tc_node(p0, p1, D, h, v, w0, b0, g, bb, w2, b2):
    return pl.pallas_call(
        _tc_node_body,
        out_shape=jax.ShapeDtypeStruct((N, H), _f32),
    )(p0, p1, D, h, v, w0, b0, g, bb, w2, b2)


def _tc_head_body(h_ref, w0_ref, b0_ref, g_ref, bb_ref, w1_ref, b1_ref, y_ref,
                  loss_ref, pred_ref):
    t = jnp.dot(h_ref[...], w0_ref[...], preferred_element_type=_f32) + b0_ref[...]
    t = _leaky(t)
    m = jnp.mean(t, axis=0, keepdims=True)
    ct = t - m
    var = jnp.mean(ct * ct, axis=0, keepdims=True)
    tn = ct * lax.rsqrt(var + EPS) * g_ref[...] + bb_ref[...]
    pred = jnp.dot(tn, w1_ref[...], preferred_element_type=_f32) + b1_ref[...]
    pred_ref[...] = pred
    r = pred - y_ref[...]
    loss_ref[...] = jnp.mean(r * r).reshape(1, 1)


def _tc_head(h, w0, b0, g, bb, w1, b1, y):
    return pl.pallas_call(
        _tc_head_body,
        out_shape=(jax.ShapeDtypeStruct((1, 1), _f32),
                   jax.ShapeDtypeStruct((N, 1), _f32)),
    )(h, w0, b0, g, bb, w1, b1, y)


def _row(v):
    return v.reshape(1, -1).astype(_f32)


def kernel(x, edge_attr, edge_index, y, params):
    x = x.astype(_f32)
    edge_attr = edge_attr.astype(_f32)
    y = y.astype(_f32)
    src = edge_index[0].astype(jnp.int32)
    dst = edge_index[1].astype(jnp.int32)

    p = params
    wfb = p["f_b"]["W"].astype(_f32)
    wfb_pad = jnp.zeros((7, CP, H), _f32).at[:, :STEPS, :].set(
        wfb.reshape(7, STEPS, H)).reshape(7 * CP, H)
    w1m_all = jnp.concatenate(
        [p["layers"][l]["f_e"][0]["W"][H:2 * H, :] for l in range(3)], axis=0)
    b1_all = jnp.stack([p["layers"][l]["f_e"][0]["b"] for l in range(3)], axis=0)

    ea_aug = jnp.concatenate([edge_attr, jnp.ones((E, 1), _f32)], axis=1)
    centers = (jnp.arange(CP, dtype=_f32) / (STEPS - 1.0))
    sel = jnp.zeros((8, 7 * CP), _f32)
    for a in range(7):
        sel = sel.at[a, a * CP:(a + 1) * CP].set(1.0)
        sel = sel.at[7, a * CP:(a + 1) * CP].set(-centers)
    ea4d = edge_attr[:, 3].reshape(NBLK, RB // H, H)
    g0, g1, g2, coef2 = _tc_pre(ea_aug, ea4d, sel, wfb_pad, _row(p["f_b"]["b"]),
                                w1m_all.astype(_f32), b1_all.astype(_f32))
    coef = coef2.reshape(E, 1)
    gmids = (g0, g1, g2)

    h = _tc_init(x, p["f_x"]["W"].astype(_f32), _row(p["f_x"]["b"]))

    for l in range(3):
        lp = p["layers"][l]
        w1 = lp["f_e"][0]["W"].astype(_f32)
        A, C, D = _tc_nodepre(h, w1[:H, :], w1[2 * H:, :],
                              lp["f_d"]["W"].astype(_f32), _row(lp["f_d"]["b"]))
        y1, st1p = _sc_p1(A, C, gmids[l], src, dst)
        st1 = st1p.reshape(NW, 2, H).sum(axis=0)
        y2, st2 = _tc_p2(st1, _row(lp["f_e"][1]["g"]), _row(lp["f_e"][1]["b"]),
                         lp["f_e"][2]["W"].astype(_f32), _row(lp["f_e"][2]["b"]),
                         y1)
        cond = _tc_p3a(st2, _row(lp["f_e"][3]["g"]), _row(lp["f_e"][3]["b"]),
                       lp["f_e"][4]["W"].astype(_f32), _row(lp["f_e"][4]["b"]),
                       y2, coef)
        parts = _sc_p3(cond, D, src, dst)
        h = _tc_node(parts[:N], parts[NPAD:NPAD + N], D, h, lp["v"].astype(_f32),
                     lp["f_n"][0]["W"].astype(_f32), _row(lp["f_n"][0]["b"]),
                     _row(lp["f_n"][1]["g"]), _row(lp["f_n"][1]["b"]),
                     lp["f_n"][2]["W"].astype(_f32), _row(lp["f_n"][2]["b"]))

    ft = p["f_target"]
    loss, pred = _tc_head(h, ft[0]["W"].astype(_f32), _row(ft[0]["b"]),
                          _row(ft[1]["g"]), _row(ft[1]["b"]),
                          ft[2]["W"].astype(_f32), _row(ft[2]["b"]), y)
    return loss[0, 0], pred


# speedup vs baseline: 1.1528x; 1.0510x over previous
import functools

import jax
import jax.numpy as jnp
from jax import lax
from jax.experimental import pallas as pl
from jax.experimental.pallas import tpu as pltpu
from jax.experimental.pallas import tpu_sc as plsc

N = 10000
E = 160000
H = 128
STEPS = 50
CP = 64
EPS = 1e-5
SLOPE = 0.01
E_F = float(E)

RB = 3200
NBLK = E // RB
K = 128
NCHUNK = E // K
K3 = 80
NCHUNK3 = E // K3
NC, NS = 2, 16
NW = NC * NS
NPAD = 10240
ROWS_PER_TILE = NPAD // NS

_f32 = jnp.float32


def _leaky(x):
    return jnp.maximum(x, x * SLOPE)


def _sc_p1_body(a_hbm, c_hbm, g_hbm, src_hbm, dst_hbm, y_hbm, st_hbm,
                idx_s0, idx_d0, a_v0, c_v0, g_v0,
                idx_s1, idx_d1, a_v1, c_v1, g_v1,
                st_v, semg0, semg1, semy0, semy1):
    cid = lax.axis_index("c")
    sid = lax.axis_index("s")
    w = cid * NS + sid
    nchunks = (NCHUNK - w + NW - 1) // NW
    bufs = ((idx_s0, idx_d0, a_v0, c_v0, g_v0, semg0, semy0),
            (idx_s1, idx_d1, a_v1, c_v1, g_v1, semg1, semy1))
    zero = jnp.zeros((16,), _f32)
    init = (zero,) * 16
    for k in range(16):
        st_v[pl.ds(k * 16, 16)] = zero

    def cbase(k):
        return (w + k * NW) * K

    def fire(k, buf):
        idx_s, idx_d, a_v, c_v, g_v, semg, _ = buf
        base = cbase(k)
        pltpu.sync_copy(src_hbm.at[pl.ds(base, K)], idx_s)
        pltpu.sync_copy(dst_hbm.at[pl.ds(base, K)], idx_d)
        pltpu.async_copy(a_hbm.at[idx_s], a_v, semg)
        pltpu.async_copy(c_hbm.at[idx_d], c_v, semg)
        pltpu.async_copy(g_hbm.at[pl.ds(base, K)], g_v, semg)

    def process(k, b):
        idx_s, idx_d, a_v, c_v, g_v, semg, semy = bufs[b]
        idx_so, _, a_vo, _, _, _, semyo = bufs[1 - b]

        @pl.when(k >= 1)
        def _():
            pltpu.make_async_copy(y_hbm.at[pl.ds(0, K)], a_vo, semyo).wait()

        @pl.when(k + 1 < nchunks)
        def _():
            fire(k + 1, bufs[1 - b])

        base = cbase(k)
        pltpu.make_async_copy(a_hbm.at[idx_s], a_v, semg).wait()
        pltpu.make_async_copy(c_hbm.at[idx_d], c_v, semg).wait()
        pltpu.make_async_copy(g_hbm.at[pl.ds(base, K)], g_v, semg).wait()

        def row_body(i, st_in):
            out = list(st_in)
            for kk in range(8):
                sl = pl.ds(kk * 16, 16)
                p = a_v[i, sl] + c_v[i, sl] + g_v[i, sl]
                yv = jnp.maximum(p, p * SLOPE)
                a_v[i, sl] = yv
                out[kk] = out[kk] + yv
                out[8 + kk] = out[8 + kk] + yv * yv
            return tuple(out)

        st = lax.fori_loop(0, K, row_body, init)
        for kk in range(16):
            sl = pl.ds(kk * 16, 16)
            st_v[sl] = st_v[sl] + st[kk]
        pltpu.async_copy(a_v, y_hbm.at[pl.ds(base, K)], semy)

    fire(0, bufs[0])

    def pair_body(p, carry):
        process(2 * p, 0)

        @pl.when(2 * p + 1 < nchunks)
        def _():
            process(2 * p + 1, 1)

        return carry

    lax.fori_loop(0, (nchunks + 1) // 2, pair_body, 0)
    last = (nchunks - 1) % 2

    @pl.when(last == 0)
    def _():
        pltpu.make_async_copy(y_hbm.at[pl.ds(0, K)], a_v0, semy0).wait()

    @pl.when(last == 1)
    def _():
        pltpu.make_async_copy(y_hbm.at[pl.ds(0, K)], a_v1, semy1).wait()

    pltpu.sync_copy(st_v, st_hbm.at[pl.ds(w * 2 * H, 2 * H)])


def _sc_p1(A, C, G, src, dst):
    mesh = plsc.VectorSubcoreMesh(core_axis_name="c", subcore_axis_name="s")
    buf_set = [
        pltpu.VMEM((K,), jnp.int32),
        pltpu.VMEM((K,), jnp.int32),
        pltpu.VMEM((K, H), _f32),
        pltpu.VMEM((K, H), _f32),
        pltpu.VMEM((K, H), _f32),
    ]
    f = pl.kernel(
        _sc_p1_body,
        out_type=(jax.ShapeDtypeStruct((E, H), _f32),
                  jax.ShapeDtypeStruct((NW * 2 * H,), _f32)),
        mesh=mesh,
        scratch_types=buf_set + buf_set + [
            pltpu.VMEM((2 * H,), _f32),
            pltpu.SemaphoreType.DMA,
            pltpu.SemaphoreType.DMA,
            pltpu.SemaphoreType.DMA,
            pltpu.SemaphoreType.DMA,
        ],
    )
    return f(A, C, G, src, dst)


def _sc_p3_body(cond_hbm, d_hbm, src_hbm, dst_hbm, out_hbm,
                idx_s0, idx_d0, m_v0, d_v0,
                idx_s1, idx_d1, m_v1, d_v1,
                acc, semg0, semg1, semsc0, semsc1):
    cid = lax.axis_index("c")
    sid = lax.axis_index("s")
    w = cid * NS + sid
    bufs = ((idx_s0, idx_d0, m_v0, d_v0, semg0, semsc0),
            (idx_s1, idx_d1, m_v1, d_v1, semg1, semsc1))

    def zrow(i, _):
        for k in range(8):
            m_v0[i, pl.ds(k * 16, 16)] = jnp.zeros((16,), _f32)
        return 0

    lax.fori_loop(0, K3, zrow, 0)
    base_r = sid * ROWS_PER_TILE
    for t in range(ROWS_PER_TILE // K3):
        pltpu.sync_copy(m_v0, acc.at[pl.ds(base_r + t * K3, K3)])
    plsc.subcore_barrier()

    nchunks = (NCHUNK3 - w + NW - 1) // NW

    def cbase(k):
        return (w + k * NW) * K3

    def fire(k, buf):
        idx_s, idx_d, m_v, d_v, semg, _ = buf
        base = cbase(k)
        pltpu.sync_copy(src_hbm.at[pl.ds(base, K3)], idx_s)
        pltpu.sync_copy(dst_hbm.at[pl.ds(base, K3)], idx_d)
        pltpu.async_copy(d_hbm.at[idx_s], d_v, semg)
        pltpu.async_copy(cond_hbm.at[pl.ds(base, K3)], m_v, semg)

    def process(k, b):
        idx_s, idx_d, m_v, d_v, semg, semsc = bufs[b]
        _, idx_do, m_vo, _, _, semsco = bufs[1 - b]

        @pl.when(k >= 1)
        def _():
            pltpu.make_async_copy(m_vo, acc.at[idx_do], semsco).wait()

        @pl.when(k + 1 < nchunks)
        def _():
            fire(k + 1, bufs[1 - b])

        base = cbase(k)
        pltpu.make_async_copy(d_hbm.at[idx_s], d_v, semg).wait()
        pltpu.make_async_copy(cond_hbm.at[pl.ds(base, K3)], m_v, semg).wait()

        def row_body(i, _):
            for kk in range(8):
                sl = pl.ds(kk * 16, 16)
                m_v[i, sl] = m_v[i, sl] * d_v[i, sl]
            return 0

        lax.fori_loop(0, K3, row_body, 0)
        pltpu.async_copy(m_v, acc.at[idx_d], semsc, add=True)

    fire(0, bufs[0])

    def pair_body(p, carry):
        process(2 * p, 0)

        @pl.when(2 * p + 1 < nchunks)
        def _():
            process(2 * p + 1, 1)

        return carry

    lax.fori_loop(0, (nchunks + 1) // 2, pair_body, 0)
    last = (nchunks - 1) % 2

    @pl.when(last == 0)
    def _():
        pltpu.make_async_copy(m_v0, acc.at[idx_d0], semsc0).wait()

    @pl.when(last == 1)
    def _():
        pltpu.make_async_copy(m_v1, acc.at[idx_d1], semsc1).wait()

    plsc.subcore_barrier()

    out_base = cid * NPAD + base_r
    for t in range(ROWS_PER_TILE // K3):
        pltpu.sync_copy(acc.at[pl.ds(base_r + t * K3, K3)], m_v0)
        pltpu.sync_copy(m_v0, out_hbm.at[pl.ds(out_base + t * K3, K3)])


def _sc_p3(cond, D, src, dst):
    mesh = plsc.VectorSubcoreMesh(core_axis_name="c", subcore_axis_name="s")
    buf_set = [
        pltpu.VMEM((K3,), jnp.int32),
        pltpu.VMEM((K3,), jnp.int32),
        pltpu.VMEM((K3, H), _f32),
        pltpu.VMEM((K3, H), _f32),
    ]
    f = pl.kernel(
        _sc_p3_body,
        out_type=jax.ShapeDtypeStruct((NC * NPAD, H), _f32),
        mesh=mesh,
        scratch_types=buf_set + buf_set + [
            pltpu.VMEM_SHARED((NPAD, H), _f32),
            pltpu.SemaphoreType.DMA,
            pltpu.SemaphoreType.DMA,
            pltpu.SemaphoreType.DMA,
            pltpu.SemaphoreType.DMA,
        ],
    )
    return f(cond, D, src, dst)


def _tc_pre_body(ea_ref, ea4_ref, wfbp_ref, bfb_ref, w1m_ref, b1_ref,
                 g0_ref, g1_ref, g2_ref, coef_ref):
    ea = ea_ref[...]
    centers = lax.broadcasted_iota(jnp.int32, (1, CP), 1).astype(_f32) / (STEPS - 1.0)
    g = jnp.zeros((RB, H), _f32) + bfb_ref[...]
    for a in range(7):
        col = ea[:, a:a + 1]
        dexp = jnp.exp(-((col - centers) ** 2) * STEPS)
        g = g + jnp.dot(dexp, wfbp_ref[pl.ds(a * CP, CP), :],
                        preferred_element_type=_f32)
    for l, gref in enumerate((g0_ref, g1_ref, g2_ref)):
        gref[...] = jnp.dot(g, w1m_ref[pl.ds(l * H, H), :],
                            preferred_element_type=_f32) + b1_ref[l:l + 1, :]
    coef_ref[...] = jnp.cos(1.5707963267948966 * ea4_ref[...])


def _tc_pre(edge_attr, ea4d, wfb_pad, bfb, w1m_all, b1_all):
    out_shape = (jax.ShapeDtypeStruct((E, H), _f32),) * 3 + (
        jax.ShapeDtypeStruct((NBLK, RB // H, H), _f32),)
    full = lambda shp: pl.BlockSpec(shp, lambda i: (0, 0))
    return pl.pallas_call(
        _tc_pre_body,
        grid=(NBLK,),
        in_specs=[pl.BlockSpec((RB, 7), lambda i: (i, 0)),
                  pl.BlockSpec((1, RB // H, H), lambda i: (i, 0, 0)),
                  full((7 * CP, H)), full((1, H)),
                  full((3 * H, H)), full((3, H))],
        out_specs=[pl.BlockSpec((RB, H), lambda i: (i, 0))] * 3 +
                  [pl.BlockSpec((1, RB // H, H), lambda i: (i, 0, 0))],
        out_shape=out_shape,
    )(edge_attr, ea4d, wfb_pad, bfb, w1m_all, b1_all)


def _tc_init_body(x_ref, w_ref, b_ref, wa_ref, wc_ref, wd_ref, bd_ref,
                  h_ref, a_ref, c_ref, d_ref):
    h = jnp.dot(x_ref[...], w_ref[...], preferred_element_type=_f32) + b_ref[...]
    h_ref[...] = h
    a_ref[...] = jnp.dot(h, wa_ref[...], preferred_element_type=_f32)
    c_ref[...] = jnp.dot(h, wc_ref[...], preferred_element_type=_f32)
    d_ref[...] = jnp.dot(h, wd_ref[...], preferred_element_type=_f32) + bd_ref[...]


def _tc_init(x, w, b, wa, wc, wd, bd):
    return pl.pallas_call(
        _tc_init_body,
        out_shape=(jax.ShapeDtypeStruct((N, H), _f32),) * 4,
    )(x, w, b, wa, wc, wd, bd)


def _tc_nodepre_body(h_ref, wa_ref, wc_ref, wd_ref, bd_ref, a_ref, c_ref, d_ref):
    h = h_ref[...]
    a_ref[...] = jnp.dot(h, wa_ref[...], preferred_element_type=_f32)
    c_ref[...] = jnp.dot(h, wc_ref[...], preferred_element_type=_f32)
    d_ref[...] = jnp.dot(h, wd_ref[...], preferred_element_type=_f32) + bd_ref[...]


def _tc_nodepre(h, wa, wc, wd, bd):
    return pl.pallas_call(
        _tc_nodepre_body,
        out_shape=(jax.ShapeDtypeStruct((N, H), _f32),) * 3,
    )(h, wa, wc, wd, bd)


def _bn_scale_shift(st_row_sum, st_row_sq, g, b):
    m = st_row_sum / E_F
    var = st_row_sq / E_F - m * m
    sc = g * lax.rsqrt(var + EPS)
    sh = b - m * sc
    return sc, sh


def _tc_p2_body(st_ref, bng_ref, bnb_ref, w2_ref, b2_ref, y1_ref,
                y2_ref, st2_ref):
    i = pl.program_id(0)
    straw = jnp.sum(st_ref[...], axis=0, keepdims=True)
    sc, sh = _bn_scale_shift(straw[:, 0:H], straw[:, H:2 * H],
                             bng_ref[...], bnb_ref[...])
    y1n = y1_ref[...] * sc + sh
    u = jnp.dot(y1n, w2_ref[...], preferred_element_type=_f32) + b2_ref[...]
    y2 = jnp.maximum(u, u * SLOPE)
    y2_ref[...] = y2

    @pl.when(i == 0)
    def _():
        st2_ref[...] = jnp.zeros_like(st2_ref)

    st2_ref[0:1, :] += jnp.sum(y2, axis=0, keepdims=True)
    st2_ref[1:2, :] += jnp.sum(y2 * y2, axis=0, keepdims=True)


def _tc_p2(st1, bng, bnb, w2, b2, y1):
    full = lambda shp: pl.BlockSpec(shp, lambda i: (0, 0))
    return pl.pallas_call(
        _tc_p2_body,
        grid=(NBLK,),
        in_specs=[full((NW, 2 * H)), full((1, H)), full((1, H)),
                  full((H, H)), full((1, H)),
                  pl.BlockSpec((RB, H), lambda i: (i, 0))],
        out_specs=[pl.BlockSpec((RB, H), lambda i: (i, 0)),
                   pl.BlockSpec((8, H), lambda i: (0, 0))],
        out_shape=(jax.ShapeDtypeStruct((E, H), _f32),
                   jax.ShapeDtypeStruct((8, H), _f32)),
    )(st1, bng, bnb, w2, b2, y1)


def _tc_p3a_body(st2_ref, bng_ref, bnb_ref, w3_ref, b3_ref, y2_ref, coef_ref,
                 cond_ref):
    sc, sh = _bn_scale_shift(st2_ref[0:1, :], st2_ref[1:2, :],
                             bng_ref[...], bnb_ref[...])
    y2n = y2_ref[...] * sc + sh
    fe = jnp.dot(y2n, w3_ref[...], preferred_element_type=_f32) + b3_ref[...]
    cond_ref[...] = fe * coef_ref[...]


def _tc_p3a(st2, bng, bnb, w3, b3, y2, coef):
    full = lambda shp: pl.BlockSpec(shp, lambda i: (0, 0))
    return pl.pallas_call(
        _tc_p3a_body,
        grid=(NBLK,),
        in_specs=[full((8, H)), full((1, H)), full((1, H)),
                  full((H, H)), full((1, H)),
                  pl.BlockSpec((RB, H), lambda i: (i, 0)),
                  pl.BlockSpec((RB, 1), lambda i: (i, 0))],
        out_specs=pl.BlockSpec((RB, H), lambda i: (i, 0)),
        out_shape=jax.ShapeDtypeStruct((E, H), _f32),
    )(st2, bng, bnb, w3, b3, y2, coef)


def _node_update(p0_ref, p1_ref, d_ref, h_ref, v_ref, w0_ref, b0_ref,
                 g_ref, bb_ref, w2_ref, b2_ref):
    z = (v_ref[...] * d_ref[...] + p0_ref[0, :N, :] + p1_ref[0, :N, :])
    z1 = jnp.dot(z, w0_ref[...], preferred_element_type=_f32) + b0_ref[...]
    z1 = _leaky(z1)
    m = jnp.mean(z1, axis=0, keepdims=True)
    cz = z1 - m
    var = jnp.mean(cz * cz, axis=0, keepdims=True)
    z1n = cz * lax.rsqrt(var + EPS) * g_ref[...] + bb_ref[...]
    z2 = jnp.dot(z1n, w2_ref[...], preferred_element_type=_f32) + b2_ref[...]
    return z2 + h_ref[...]


def _tc_node_body(p0_ref, p1_ref, d_ref, h_ref, v_ref, w0_ref, b0_ref,
                  g_ref, bb_ref, w2_ref, b2_ref, ho_ref):
    ho_ref[...] = _node_update(p0_ref, p1_ref, d_ref, h_ref, v_ref, w0_ref,
                               b0_ref, g_ref, bb_ref, w2_ref, b2_ref)


def _tc_nodef_body(p0_ref, p1_ref, d_ref, h_ref, v_ref, w0_ref, b0_ref,
                   g_ref, bb_ref, w2_ref, b2_ref,
                   wa_ref, wc_ref, wd_ref, bd_ref,
                   ho_ref, a_ref, c_ref, dn_ref):
    hn = _node_update(p0_ref, p1_ref, d_ref, h_ref, v_ref, w0_ref,
                      b0_ref, g_ref, bb_ref, w2_ref, b2_ref)
    ho_ref[...] = hn
    a_ref[...] = jnp.dot(hn, wa_ref[...], preferred_element_type=_f32)
    c_ref[...] = jnp.dot(hn, wc_ref[...], preferred_element_type=_f32)
    dn_ref[...] = jnp.dot(hn, wd_ref[...], preferred_element_type=_f32) + bd_ref[...]


def _fs(shp):
    return pl.BlockSpec(shp, lambda i: tuple(0 for _ in shp))


def _parts_specs():
    return [pl.BlockSpec((1, NPAD, H), lambda i: (0, 0, 0)),
            pl.BlockSpec((1, NPAD, H), lambda i: (1, 0, 0))]


_NODE_TAIL = [(N, H), (N, H), (1, H), (H, H), (1, H), (1, H), (1, H),
              (H, H), (1, H)]
_PROJ_TAIL = [(H, H), (H, H), (H, H), (1, H)]


def _tc_node(parts, D, h, v, w0, b0, g, bb, w2, b2):
    return pl.pallas_call(
        _tc_node_body,
        grid=(1,),
        in_specs=_parts_specs() + [_fs(s) for s in _NODE_TAIL],
        out_specs=_fs((N, H)),
        out_shape=jax.ShapeDtypeStruct((N, H), _f32),
    )(parts, parts, D, h, v, w0, b0, g, bb, w2, b2)


def _tc_nodef(parts, D, h, v, w0, b0, g, bb, w2, b2, wa, wc, wd, bd):
    return pl.pallas_call(
        _tc_nodef_body,
        grid=(1,),
        in_specs=_parts_specs() + [_fs(s) for s in _NODE_TAIL + _PROJ_TAIL],
        out_specs=[_fs((N, H))] * 4,
        out_shape=(jax.ShapeDtypeStruct((N, H), _f32),) * 4,
    )(parts, parts, D, h, v, w0, b0, g, bb, w2, b2, wa, wc, wd, bd)


def _tc_head_body(h_ref, w0_ref, b0_ref, g_ref, bb_ref, w1_ref, b1_ref, y_ref,
                  loss_ref, pred_ref):
    t = jnp.dot(h_ref[...], w0_ref[...], preferred_element_type=_f32) + b0_ref[...]
    t = _leaky(t)
    m = jnp.mean(t, axis=0, keepdims=True)
    ct = t - m
    var = jnp.mean(ct * ct, axis=0, keepdims=True)
    tn = ct * lax.rsqrt(var + EPS) * g_ref[...] + bb_ref[...]
    pred = jnp.dot(tn, w1_ref[...], preferred_element_type=_f32) + b1_ref[...]
    pred_ref[...] = pred
    r = pred - y_ref[...]
    loss_ref[...] = jnp.mean(r * r).reshape(1, 1)


def _tc_head(h, w0, b0, g, bb, w1, b1, y):
    return pl.pallas_call(
        _tc_head_body,
        out_shape=(jax.ShapeDtypeStruct((1, 1), _f32),
                   jax.ShapeDtypeStruct((N, 1), _f32)),
    )(h, w0, b0, g, bb, w1, b1, y)


def _row(v):
    return v.reshape(1, -1).astype(_f32)


def kernel(x, edge_attr, edge_index, y, params):
    x = x.astype(_f32)
    edge_attr = edge_attr.astype(_f32)
    y = y.astype(_f32)
    src = edge_index[0].astype(jnp.int32)
    dst = edge_index[1].astype(jnp.int32)

    p = params
    wfb = p["f_b"]["W"].astype(_f32)
    wfb_pad = jnp.zeros((7, CP, H), _f32).at[:, :STEPS, :].set(
        wfb.reshape(7, STEPS, H)).reshape(7 * CP, H)
    w1m_all = jnp.concatenate(
        [p["layers"][l]["f_e"][0]["W"][H:2 * H, :] for l in range(3)], axis=0)
    b1_all = jnp.stack([p["layers"][l]["f_e"][0]["b"] for l in range(3)], axis=0)

    ea4d = edge_attr[:, 3].reshape(NBLK, RB // H, H)
    g0, g1, g2, coef2 = _tc_pre(edge_attr, ea4d, wfb_pad, _row(p["f_b"]["b"]),
                                w1m_all.astype(_f32), b1_all.astype(_f32))
    coef = coef2.reshape(E, 1)
    gmids = (g0, g1, g2)

    def proj_w(l):
        lp = p["layers"][l]
        w1 = lp["f_e"][0]["W"].astype(_f32)
        return (w1[:H, :], w1[2 * H:, :],
                lp["f_d"]["W"].astype(_f32), _row(lp["f_d"]["b"]))

    h, A, C, D = _tc_init(x, p["f_x"]["W"].astype(_f32), _row(p["f_x"]["b"]),
                          *proj_w(0))

    for l in range(3):
        lp = p["layers"][l]
        y1, st1p = _sc_p1(A, C, gmids[l], src, dst)
        y2, st2 = _tc_p2(st1p.reshape(NW, 2 * H),
                         _row(lp["f_e"][1]["g"]), _row(lp["f_e"][1]["b"]),
                         lp["f_e"][2]["W"].astype(_f32), _row(lp["f_e"][2]["b"]),
                         y1)
        cond = _tc_p3a(st2, _row(lp["f_e"][3]["g"]), _row(lp["f_e"][3]["b"]),
                       lp["f_e"][4]["W"].astype(_f32), _row(lp["f_e"][4]["b"]),
                       y2, coef)
        parts = _sc_p3(cond, D, src, dst).reshape(2, NPAD, H)
        node_args = (parts, D, h, lp["v"].astype(_f32),
                     lp["f_n"][0]["W"].astype(_f32), _row(lp["f_n"][0]["b"]),
                     _row(lp["f_n"][1]["g"]), _row(lp["f_n"][1]["b"]),
                     lp["f_n"][2]["W"].astype(_f32), _row(lp["f_n"][2]["b"]))
        if l < 2:
            h, A, C, D = _tc_nodef(*node_args, *proj_w(l + 1))
        else:
            h = _tc_node(*node_args)

    ft = p["f_target"]
    loss, pred = _tc_head(h, ft[0]["W"].astype(_f32), _row(ft[0]["b"]),
                          _row(ft[1]["g"]), _row(ft[1]["b"]),
                          ft[2]["W"].astype(_f32), _row(ft[2]["b"]), y)
    return loss[0, 0], pred


# speedup vs baseline: 1.1981x; 1.0392x over previous
import functools

import jax
import jax.numpy as jnp
from jax import lax
from jax.experimental import pallas as pl
from jax.experimental.pallas import tpu as pltpu
from jax.experimental.pallas import tpu_sc as plsc

N = 10000
E = 160000
H = 128
STEPS = 50
CP = 64
EPS = 1e-5
SLOPE = 0.01
E_F = float(E)

RB = 3200
NBLK = E // RB
K = 128
NCHUNK = E // K
K3 = 80
NCHUNK3 = E // K3
NC, NS = 2, 16
NW = NC * NS
NPAD = 10240
ROWS_PER_TILE = NPAD // NS

_f32 = jnp.float32


def _leaky(x):
    return jnp.maximum(x, x * SLOPE)


def _sc_p1_body(a_hbm, c_hbm, g_hbm, src_hbm, dst_hbm, y_hbm, st_hbm,
                idx_s0, idx_d0, a_v0, c_v0, g_v0,
                idx_s1, idx_d1, a_v1, c_v1, g_v1,
                st_v, semg0, semg1, semy0, semy1, semi0, semi1):
    cid = lax.axis_index("c")
    sid = lax.axis_index("s")
    w = cid * NS + sid
    nchunks = (NCHUNK - w + NW - 1) // NW
    bufs = ((idx_s0, idx_d0, a_v0, c_v0, g_v0, semg0, semy0, semi0),
            (idx_s1, idx_d1, a_v1, c_v1, g_v1, semg1, semy1, semi1))
    zero = jnp.zeros((16,), _f32)
    init = (zero,) * 16
    for k in range(16):
        st_v[pl.ds(k * 16, 16)] = zero

    def cbase(k):
        return (w + k * NW) * K

    def fire_idx(k, buf):
        idx_s, idx_d = buf[0], buf[1]
        semi = buf[7]
        base = cbase(k)
        pltpu.async_copy(src_hbm.at[pl.ds(base, K)], idx_s, semi)
        pltpu.async_copy(dst_hbm.at[pl.ds(base, K)], idx_d, semi)

    def wait_idx(k, buf):
        idx_s, idx_d = buf[0], buf[1]
        semi = buf[7]
        base = cbase(k)
        pltpu.make_async_copy(src_hbm.at[pl.ds(base, K)], idx_s, semi).wait()
        pltpu.make_async_copy(dst_hbm.at[pl.ds(base, K)], idx_d, semi).wait()

    def fire_gather(k, buf):
        idx_s, idx_d, a_v, c_v, g_v, semg = buf[:6]
        base = cbase(k)
        pltpu.async_copy(a_hbm.at[idx_s], a_v, semg)
        pltpu.async_copy(c_hbm.at[idx_d], c_v, semg)
        pltpu.async_copy(g_hbm.at[pl.ds(base, K)], g_v, semg)

    def process(k, b):
        idx_s, idx_d, a_v, c_v, g_v, semg, semy, _ = bufs[b]
        a_vo = bufs[1 - b][2]
        semyo = bufs[1 - b][6]

        @pl.when(k >= 1)
        def _():
            pltpu.make_async_copy(y_hbm.at[pl.ds(0, K)], a_vo, semyo).wait()

        @pl.when(k + 1 < nchunks)
        def _():
            wait_idx(k + 1, bufs[1 - b])
            fire_gather(k + 1, bufs[1 - b])

        base = cbase(k)
        pltpu.make_async_copy(a_hbm.at[idx_s], a_v, semg).wait()
        pltpu.make_async_copy(c_hbm.at[idx_d], c_v, semg).wait()
        pltpu.make_async_copy(g_hbm.at[pl.ds(base, K)], g_v, semg).wait()

        @pl.when(k + 2 < nchunks)
        def _():
            fire_idx(k + 2, bufs[b])

        def row_body(i, st_in):
            out = list(st_in)
            for kk in range(8):
                sl = pl.ds(kk * 16, 16)
                p = a_v[i, sl] + c_v[i, sl] + g_v[i, sl]
                yv = jnp.maximum(p, p * SLOPE)
                a_v[i, sl] = yv
                out[kk] = out[kk] + yv
                out[8 + kk] = out[8 + kk] + yv * yv
            return tuple(out)

        st = lax.fori_loop(0, K, row_body, init)
        for kk in range(16):
            sl = pl.ds(kk * 16, 16)
            st_v[sl] = st_v[sl] + st[kk]
        pltpu.async_copy(a_v, y_hbm.at[pl.ds(base, K)], semy)

    fire_idx(0, bufs[0])
    wait_idx(0, bufs[0])
    fire_gather(0, bufs[0])

    @pl.when(1 < nchunks)
    def _():
        fire_idx(1, bufs[1])

    def pair_body(p, carry):
        process(2 * p, 0)

        @pl.when(2 * p + 1 < nchunks)
        def _():
            process(2 * p + 1, 1)

        return carry

    lax.fori_loop(0, (nchunks + 1) // 2, pair_body, 0)
    last = (nchunks - 1) % 2

    @pl.when(last == 0)
    def _():
        pltpu.make_async_copy(y_hbm.at[pl.ds(0, K)], a_v0, semy0).wait()

    @pl.when(last == 1)
    def _():
        pltpu.make_async_copy(y_hbm.at[pl.ds(0, K)], a_v1, semy1).wait()

    pltpu.sync_copy(st_v, st_hbm.at[pl.ds(w * 2 * H, 2 * H)])


def _sc_p1(A, C, G, src, dst):
    mesh = plsc.VectorSubcoreMesh(core_axis_name="c", subcore_axis_name="s")
    buf_set = [
        pltpu.VMEM((K,), jnp.int32),
        pltpu.VMEM((K,), jnp.int32),
        pltpu.VMEM((K, H), _f32),
        pltpu.VMEM((K, H), _f32),
        pltpu.VMEM((K, H), _f32),
    ]
    f = pl.kernel(
        _sc_p1_body,
        out_type=(jax.ShapeDtypeStruct((E, H), _f32),
                  jax.ShapeDtypeStruct((NW * 2 * H,), _f32)),
        mesh=mesh,
        scratch_types=buf_set + buf_set + [
            pltpu.VMEM((2 * H,), _f32),
            pltpu.SemaphoreType.DMA,
            pltpu.SemaphoreType.DMA,
            pltpu.SemaphoreType.DMA,
            pltpu.SemaphoreType.DMA,
            pltpu.SemaphoreType.DMA,
            pltpu.SemaphoreType.DMA,
        ],
    )
    return f(A, C, G, src, dst)


def _sc_p3_body(cond_hbm, d_hbm, src_hbm, dst_hbm, out_hbm,
                idx_s0, idx_d0, m_v0, d_v0,
                idx_s1, idx_d1, m_v1, d_v1,
                acc, semg0, semg1, semsc0, semsc1):
    cid = lax.axis_index("c")
    sid = lax.axis_index("s")
    w = cid * NS + sid
    bufs = ((idx_s0, idx_d0, m_v0, d_v0, semg0, semsc0),
            (idx_s1, idx_d1, m_v1, d_v1, semg1, semsc1))

    def zrow(i, _):
        for k in range(8):
            m_v0[i, pl.ds(k * 16, 16)] = jnp.zeros((16,), _f32)
        return 0

    lax.fori_loop(0, K3, zrow, 0)
    base_r = sid * ROWS_PER_TILE
    for t in range(ROWS_PER_TILE // K3):
        pltpu.sync_copy(m_v0, acc.at[pl.ds(base_r + t * K3, K3)])
    plsc.subcore_barrier()

    nchunks = (NCHUNK3 - w + NW - 1) // NW

    def cbase(k):
        return (w + k * NW) * K3

    def fire(k, buf):
        idx_s, idx_d, m_v, d_v, semg, _ = buf
        base = cbase(k)
        pltpu.sync_copy(src_hbm.at[pl.ds(base, K3)], idx_s)
        pltpu.sync_copy(dst_hbm.at[pl.ds(base, K3)], idx_d)
        pltpu.async_copy(d_hbm.at[idx_s], d_v, semg)
        pltpu.async_copy(cond_hbm.at[pl.ds(base, K3)], m_v, semg)

    def process(k, b):
        idx_s, idx_d, m_v, d_v, semg, semsc = bufs[b]
        _, idx_do, m_vo, _, _, semsco = bufs[1 - b]

        @pl.when(k >= 1)
        def _():
            pltpu.make_async_copy(m_vo, acc.at[idx_do], semsco).wait()

        @pl.when(k + 1 < nchunks)
        def _():
            fire(k + 1, bufs[1 - b])

        base = cbase(k)
        pltpu.make_async_copy(d_hbm.at[idx_s], d_v, semg).wait()
        pltpu.make_async_copy(cond_hbm.at[pl.ds(base, K3)], m_v, semg).wait()

        def row_body(i, _):
            for kk in range(8):
                sl = pl.ds(kk * 16, 16)
                m_v[i, sl] = m_v[i, sl] * d_v[i, sl]
            return 0

        lax.fori_loop(0, K3, row_body, 0)
        pltpu.async_copy(m_v, acc.at[idx_d], semsc, add=True)

    fire(0, bufs[0])

    def pair_body(p, carry):
        process(2 * p, 0)

        @pl.when(2 * p + 1 < nchunks)
        def _():
            process(2 * p + 1, 1)

        return carry

    lax.fori_loop(0, (nchunks + 1) // 2, pair_body, 0)
    last = (nchunks - 1) % 2

    @pl.when(last == 0)
    def _():
        pltpu.make_async_copy(m_v0, acc.at[idx_d0], semsc0).wait()

    @pl.when(last == 1)
    def _():
        pltpu.make_async_copy(m_v1, acc.at[idx_d1], semsc1).wait()

    plsc.subcore_barrier()

    out_base = cid * NPAD + base_r
    for t in range(ROWS_PER_TILE // K3):
        pltpu.sync_copy(acc.at[pl.ds(base_r + t * K3, K3)], m_v0)
        pltpu.sync_copy(m_v0, out_hbm.at[pl.ds(out_base + t * K3, K3)])


def _sc_p3(cond, D, src, dst):
    mesh = plsc.VectorSubcoreMesh(core_axis_name="c", subcore_axis_name="s")
    buf_set = [
        pltpu.VMEM((K3,), jnp.int32),
        pltpu.VMEM((K3,), jnp.int32),
        pltpu.VMEM((K3, H), _f32),
        pltpu.VMEM((K3, H), _f32),
    ]
    f = pl.kernel(
        _sc_p3_body,
        out_type=jax.ShapeDtypeStruct((NC * NPAD, H), _f32),
        mesh=mesh,
        scratch_types=buf_set + buf_set + [
            pltpu.VMEM_SHARED((NPAD, H), _f32),
            pltpu.SemaphoreType.DMA,
            pltpu.SemaphoreType.DMA,
            pltpu.SemaphoreType.DMA,
            pltpu.SemaphoreType.DMA,
        ],
    )
    return f(cond, D, src, dst)


def _tc_pre_body(ea_ref, ea4_ref, wfbp_ref, bfb_ref, w1m_ref, b1_ref,
                 g0_ref, g1_ref, g2_ref, coef_ref):
    ea = ea_ref[...]
    centers = lax.broadcasted_iota(jnp.int32, (1, CP), 1).astype(_f32) / (STEPS - 1.0)
    g = jnp.zeros((RB, H), _f32) + bfb_ref[...]
    for a in range(7):
        col = ea[:, a:a + 1]
        dexp = jnp.exp(-((col - centers) ** 2) * STEPS)
        g = g + jnp.dot(dexp, wfbp_ref[pl.ds(a * CP, CP), :],
                        preferred_element_type=_f32)
    for l, gref in enumerate((g0_ref, g1_ref, g2_ref)):
        gref[...] = jnp.dot(g, w1m_ref[pl.ds(l * H, H), :],
                            preferred_element_type=_f32) + b1_ref[l:l + 1, :]
    coef_ref[...] = jnp.cos(1.5707963267948966 * ea4_ref[...])


def _tc_pre(edge_attr, ea4d, wfb_pad, bfb, w1m_all, b1_all):
    out_shape = (jax.ShapeDtypeStruct((E, H), _f32),) * 3 + (
        jax.ShapeDtypeStruct((NBLK, RB // H, H), _f32),)
    full = lambda shp: pl.BlockSpec(shp, lambda i: (0, 0))
    return pl.pallas_call(
        _tc_pre_body,
        grid=(NBLK,),
        in_specs=[pl.BlockSpec((RB, 7), lambda i: (i, 0)),
                  pl.BlockSpec((1, RB // H, H), lambda i: (i, 0, 0)),
                  full((7 * CP, H)), full((1, H)),
                  full((3 * H, H)), full((3, H))],
        out_specs=[pl.BlockSpec((RB, H), lambda i: (i, 0))] * 3 +
                  [pl.BlockSpec((1, RB // H, H), lambda i: (i, 0, 0))],
        out_shape=out_shape,
    )(edge_attr, ea4d, wfb_pad, bfb, w1m_all, b1_all)


def _tc_init_body(x_ref, w_ref, b_ref, wa_ref, wc_ref, wd_ref, bd_ref,
                  h_ref, a_ref, c_ref, d_ref):
    h = jnp.dot(x_ref[...], w_ref[...], preferred_element_type=_f32) + b_ref[...]
    h_ref[...] = h
    a_ref[...] = jnp.dot(h, wa_ref[...], preferred_element_type=_f32)
    c_ref[...] = jnp.dot(h, wc_ref[...], preferred_element_type=_f32)
    d_ref[...] = jnp.dot(h, wd_ref[...], preferred_element_type=_f32) + bd_ref[...]


def _tc_init(x, w, b, wa, wc, wd, bd):
    return pl.pallas_call(
        _tc_init_body,
        out_shape=(jax.ShapeDtypeStruct((N, H), _f32),) * 4,
    )(x, w, b, wa, wc, wd, bd)


def _tc_nodepre_body(h_ref, wa_ref, wc_ref, wd_ref, bd_ref, a_ref, c_ref, d_ref):
    h = h_ref[...]
    a_ref[...] = jnp.dot(h, wa_ref[...], preferred_element_type=_f32)
    c_ref[...] = jnp.dot(h, wc_ref[...], preferred_element_type=_f32)
    d_ref[...] = jnp.dot(h, wd_ref[...], preferred_element_type=_f32) + bd_ref[...]


def _tc_nodepre(h, wa, wc, wd, bd):
    return pl.pallas_call(
        _tc_nodepre_body,
        out_shape=(jax.ShapeDtypeStruct((N, H), _f32),) * 3,
    )(h, wa, wc, wd, bd)


def _bn_scale_shift(st_row_sum, st_row_sq, g, b):
    m = st_row_sum / E_F
    var = st_row_sq / E_F - m * m
    sc = g * lax.rsqrt(var + EPS)
    sh = b - m * sc
    return sc, sh


def _tc_p2_body(st_ref, bng_ref, bnb_ref, w2_ref, b2_ref, y1_ref,
                y2_ref, st2_ref):
    i = pl.program_id(0)
    straw = jnp.sum(st_ref[...], axis=0, keepdims=True)
    sc, sh = _bn_scale_shift(straw[:, 0:H], straw[:, H:2 * H],
                             bng_ref[...], bnb_ref[...])
    y1n = y1_ref[...] * sc + sh
    u = jnp.dot(y1n, w2_ref[...], preferred_element_type=_f32) + b2_ref[...]
    y2 = jnp.maximum(u, u * SLOPE)
    y2_ref[...] = y2

    @pl.when(i == 0)
    def _():
        st2_ref[...] = jnp.zeros_like(st2_ref)

    st2_ref[0:1, :] += jnp.sum(y2, axis=0, keepdims=True)
    st2_ref[1:2, :] += jnp.sum(y2 * y2, axis=0, keepdims=True)


def _tc_p2(st1, bng, bnb, w2, b2, y1):
    full = lambda shp: pl.BlockSpec(shp, lambda i: (0, 0))
    return pl.pallas_call(
        _tc_p2_body,
        grid=(NBLK,),
        in_specs=[full((NW, 2 * H)), full((1, H)), full((1, H)),
                  full((H, H)), full((1, H)),
                  pl.BlockSpec((RB, H), lambda i: (i, 0))],
        out_specs=[pl.BlockSpec((RB, H), lambda i: (i, 0)),
                   pl.BlockSpec((8, H), lambda i: (0, 0))],
        out_shape=(jax.ShapeDtypeStruct((E, H), _f32),
                   jax.ShapeDtypeStruct((8, H), _f32)),
    )(st1, bng, bnb, w2, b2, y1)


def _tc_p3a_body(st2_ref, bng_ref, bnb_ref, w3_ref, b3_ref, y2_ref, coef_ref,
                 cond_ref):
    sc, sh = _bn_scale_shift(st2_ref[0:1, :], st2_ref[1:2, :],
                             bng_ref[...], bnb_ref[...])
    y2n = y2_ref[...] * sc + sh
    fe = jnp.dot(y2n, w3_ref[...], preferred_element_type=_f32) + b3_ref[...]
    cond_ref[...] = fe * coef_ref[...]


def _tc_p3a(st2, bng, bnb, w3, b3, y2, coef):
    full = lambda shp: pl.BlockSpec(shp, lambda i: (0, 0))
    return pl.pallas_call(
        _tc_p3a_body,
        grid=(NBLK,),
        in_specs=[full((8, H)), full((1, H)), full((1, H)),
                  full((H, H)), full((1, H)),
                  pl.BlockSpec((RB, H), lambda i: (i, 0)),
                  pl.BlockSpec((RB, 1), lambda i: (i, 0))],
        out_specs=pl.BlockSpec((RB, H), lambda i: (i, 0)),
        out_shape=jax.ShapeDtypeStruct((E, H), _f32),
    )(st2, bng, bnb, w3, b3, y2, coef)


def _node_update(p0_ref, p1_ref, d_ref, h_ref, v_ref, w0_ref, b0_ref,
                 g_ref, bb_ref, w2_ref, b2_ref):
    z = (v_ref[...] * d_ref[...] + p0_ref[0, :N, :] + p1_ref[0, :N, :])
    z1 = jnp.dot(z, w0_ref[...], preferred_element_type=_f32) + b0_ref[...]
    z1 = _leaky(z1)
    m = jnp.mean(z1, axis=0, keepdims=True)
    cz = z1 - m
    var = jnp.mean(cz * cz, axis=0, keepdims=True)
    z1n = cz * lax.rsqrt(var + EPS) * g_ref[...] + bb_ref[...]
    z2 = jnp.dot(z1n, w2_ref[...], preferred_element_type=_f32) + b2_ref[...]
    return z2 + h_ref[...]


def _tc_node_body(p0_ref, p1_ref, d_ref, h_ref, v_ref, w0_ref, b0_ref,
                  g_ref, bb_ref, w2_ref, b2_ref, ho_ref):
    ho_ref[...] = _node_update(p0_ref, p1_ref, d_ref, h_ref, v_ref, w0_ref,
                               b0_ref, g_ref, bb_ref, w2_ref, b2_ref)


def _tc_nodef_body(p0_ref, p1_ref, d_ref, h_ref, v_ref, w0_ref, b0_ref,
                   g_ref, bb_ref, w2_ref, b2_ref,
                   wa_ref, wc_ref, wd_ref, bd_ref,
                   ho_ref, a_ref, c_ref, dn_ref):
    hn = _node_update(p0_ref, p1_ref, d_ref, h_ref, v_ref, w0_ref,
                      b0_ref, g_ref, bb_ref, w2_ref, b2_ref)
    ho_ref[...] = hn
    a_ref[...] = jnp.dot(hn, wa_ref[...], preferred_element_type=_f32)
    c_ref[...] = jnp.dot(hn, wc_ref[...], preferred_element_type=_f32)
    dn_ref[...] = jnp.dot(hn, wd_ref[...], preferred_element_type=_f32) + bd_ref[...]


def _fs(shp):
    return pl.BlockSpec(shp, lambda i: tuple(0 for _ in shp))


def _parts_specs():
    return [pl.BlockSpec((1, NPAD, H), lambda i: (0, 0, 0)),
            pl.BlockSpec((1, NPAD, H), lambda i: (1, 0, 0))]


_NODE_TAIL = [(N, H), (N, H), (1, H), (H, H), (1, H), (1, H), (1, H),
              (H, H), (1, H)]
_PROJ_TAIL = [(H, H), (H, H), (H, H), (1, H)]


def _tc_node(parts, D, h, v, w0, b0, g, bb, w2, b2):
    return pl.pallas_call(
        _tc_node_body,
        grid=(1,),
        in_specs=_parts_specs() + [_fs(s) for s in _NODE_TAIL],
        out_specs=_fs((N, H)),
        out_shape=jax.ShapeDtypeStruct((N, H), _f32),
    )(parts, parts, D, h, v, w0, b0, g, bb, w2, b2)


def _tc_nodef(parts, D, h, v, w0, b0, g, bb, w2, b2, wa, wc, wd, bd):
    return pl.pallas_call(
        _tc_nodef_body,
        grid=(1,),
        in_specs=_parts_specs() + [_fs(s) for s in _NODE_TAIL + _PROJ_TAIL],
        out_specs=[_fs((N, H))] * 4,
        out_shape=(jax.ShapeDtypeStruct((N, H), _f32),) * 4,
    )(parts, parts, D, h, v, w0, b0, g, bb, w2, b2, wa, wc, wd, bd)


def _tc_head_body(h_ref, w0_ref, b0_ref, g_ref, bb_ref, w1_ref, b1_ref, y_ref,
                  loss_ref, pred_ref):
    t = jnp.dot(h_ref[...], w0_ref[...], preferred_element_type=_f32) + b0_ref[...]
    t = _leaky(t)
    m = jnp.mean(t, axis=0, keepdims=True)
    ct = t - m
    var = jnp.mean(ct * ct, axis=0, keepdims=True)
    tn = ct * lax.rsqrt(var + EPS) * g_ref[...] + bb_ref[...]
    pred = jnp.dot(tn, w1_ref[...], preferred_element_type=_f32) + b1_ref[...]
    pred_ref[...] = pred
    r = pred - y_ref[...]
    loss_ref[...] = jnp.mean(r * r).reshape(1, 1)


def _tc_head(h, w0, b0, g, bb, w1, b1, y):
    return pl.pallas_call(
        _tc_head_body,
        out_shape=(jax.ShapeDtypeStruct((1, 1), _f32),
                   jax.ShapeDtypeStruct((N, 1), _f32)),
    )(h, w0, b0, g, bb, w1, b1, y)


def _row(v):
    return v.reshape(1, -1).astype(_f32)


def kernel(x, edge_attr, edge_index, y, params):
    x = x.astype(_f32)
    edge_attr = edge_attr.astype(_f32)
    y = y.astype(_f32)
    src = edge_index[0].astype(jnp.int32)
    dst = edge_index[1].astype(jnp.int32)

    p = params
    wfb = p["f_b"]["W"].astype(_f32)
    wfb_pad = jnp.zeros((7, CP, H), _f32).at[:, :STEPS, :].set(
        wfb.reshape(7, STEPS, H)).reshape(7 * CP, H)
    w1m_all = jnp.concatenate(
        [p["layers"][l]["f_e"][0]["W"][H:2 * H, :] for l in range(3)], axis=0)
    b1_all = jnp.stack([p["layers"][l]["f_e"][0]["b"] for l in range(3)], axis=0)

    ea4d = edge_attr[:, 3].reshape(NBLK, RB // H, H)
    g0, g1, g2, coef2 = _tc_pre(edge_attr, ea4d, wfb_pad, _row(p["f_b"]["b"]),
                                w1m_all.astype(_f32), b1_all.astype(_f32))
    coef = coef2.reshape(E, 1)
    gmids = (g0, g1, g2)

    def proj_w(l):
        lp = p["layers"][l]
        w1 = lp["f_e"][0]["W"].astype(_f32)
        return (w1[:H, :], w1[2 * H:, :],
                lp["f_d"]["W"].astype(_f32), _row(lp["f_d"]["b"]))

    h, A, C, D = _tc_init(x, p["f_x"]["W"].astype(_f32), _row(p["f_x"]["b"]),
                          *proj_w(0))

    for l in range(3):
        lp = p["layers"][l]
        y1, st1p = _sc_p1(A, C, gmids[l], src, dst)
        y2, st2 = _tc_p2(st1p.reshape(NW, 2 * H),
                         _row(lp["f_e"][1]["g"]), _row(lp["f_e"][1]["b"]),
                         lp["f_e"][2]["W"].astype(_f32), _row(lp["f_e"][2]["b"]),
                         y1)
        cond = _tc_p3a(st2, _row(lp["f_e"][3]["g"]), _row(lp["f_e"][3]["b"]),
                       lp["f_e"][4]["W"].astype(_f32), _row(lp["f_e"][4]["b"]),
                       y2, coef)
        parts = _sc_p3(cond, D, src, dst).reshape(2, NPAD, H)
        node_args = (parts, D, h, lp["v"].astype(_f32),
                     lp["f_n"][0]["W"].astype(_f32), _row(lp["f_n"][0]["b"]),
                     _row(lp["f_n"][1]["g"]), _row(lp["f_n"][1]["b"]),
                     lp["f_n"][2]["W"].astype(_f32), _row(lp["f_n"][2]["b"]))
        if l < 2:
            h, A, C, D = _tc_nodef(*node_args, *proj_w(l + 1))
        else:
            h = _tc_node(*node_args)

    ft = p["f_target"]
    loss, pred = _tc_head(h, ft[0]["W"].astype(_f32), _row(ft[0]["b"]),
                          _row(ft[1]["g"]), _row(ft[1]["b"]),
                          ft[2]["W"].astype(_f32), _row(ft[2]["b"]), y)
    return loss[0, 0], pred


# speedup vs baseline: 1.2935x; 1.0797x over previous
import functools

import jax
import jax.numpy as jnp
from jax import lax
from jax.experimental import pallas as pl
from jax.experimental.pallas import tpu as pltpu
from jax.experimental.pallas import tpu_sc as plsc

N = 10000
E = 160000
H = 128
STEPS = 50
CP = 64
EPS = 1e-5
SLOPE = 0.01
E_F = float(E)

RB = 3200
NBLK = E // RB
K = 128
NCHUNK = E // K
K3 = 80
NCHUNK3 = E // K3
NC, NS = 2, 16
NW = NC * NS
NPAD = 10240
ROWS_PER_TILE = NPAD // NS

_f32 = jnp.float32


def _leaky(x):
    return jnp.maximum(x, x * SLOPE)


def _sc_p1_body(a_hbm, c_hbm, g_hbm, src_hbm, dst_hbm, y_hbm, st_hbm,
                idx_s0, idx_d0, a_v0, c_v0, g_v0,
                idx_s1, idx_d1, a_v1, c_v1, g_v1,
                st_v, semg0, semg1, semy0, semy1, semi0, semi1):
    cid = lax.axis_index("c")
    sid = lax.axis_index("s")
    w = cid * NS + sid
    nchunks = (NCHUNK - w + NW - 1) // NW
    bufs = ((idx_s0, idx_d0, a_v0, c_v0, g_v0, semg0, semy0, semi0),
            (idx_s1, idx_d1, a_v1, c_v1, g_v1, semg1, semy1, semi1))
    zero = jnp.zeros((16,), _f32)
    init = (zero,) * 16
    for k in range(16):
        st_v[pl.ds(k * 16, 16)] = zero

    def cbase(k):
        return (w + k * NW) * K

    def fire_idx(k, buf):
        idx_s, idx_d = buf[0], buf[1]
        semi = buf[7]
        base = cbase(k)
        pltpu.async_copy(src_hbm.at[pl.ds(base, K)], idx_s, semi)
        pltpu.async_copy(dst_hbm.at[pl.ds(base, K)], idx_d, semi)

    def wait_idx(k, buf):
        idx_s, idx_d = buf[0], buf[1]
        semi = buf[7]
        base = cbase(k)
        pltpu.make_async_copy(src_hbm.at[pl.ds(base, K)], idx_s, semi).wait()
        pltpu.make_async_copy(dst_hbm.at[pl.ds(base, K)], idx_d, semi).wait()

    def fire_gather(k, buf):
        idx_s, idx_d, a_v, c_v, g_v, semg = buf[:6]
        base = cbase(k)
        pltpu.async_copy(a_hbm.at[idx_s], a_v, semg)
        pltpu.async_copy(c_hbm.at[idx_d], c_v, semg)
        pltpu.async_copy(g_hbm.at[pl.ds(base, K)], g_v, semg)

    def process(k, b):
        idx_s, idx_d, a_v, c_v, g_v, semg, semy, _ = bufs[b]
        a_vo = bufs[1 - b][2]
        semyo = bufs[1 - b][6]

        @pl.when(k >= 1)
        def _():
            pltpu.make_async_copy(y_hbm.at[pl.ds(0, K)], a_vo, semyo).wait()

        @pl.when(k + 1 < nchunks)
        def _():
            wait_idx(k + 1, bufs[1 - b])
            fire_gather(k + 1, bufs[1 - b])

        base = cbase(k)
        pltpu.make_async_copy(a_hbm.at[idx_s], a_v, semg).wait()
        pltpu.make_async_copy(c_hbm.at[idx_d], c_v, semg).wait()
        pltpu.make_async_copy(g_hbm.at[pl.ds(base, K)], g_v, semg).wait()

        @pl.when(k + 2 < nchunks)
        def _():
            fire_idx(k + 2, bufs[b])

        def row_body(i, st_in):
            out = list(st_in)
            for kk in range(8):
                sl = pl.ds(kk * 16, 16)
                p = a_v[i, sl] + c_v[i, sl] + g_v[i, sl]
                yv = jnp.maximum(p, p * SLOPE)
                a_v[i, sl] = yv
                out[kk] = out[kk] + yv
                out[8 + kk] = out[8 + kk] + yv * yv
            return tuple(out)

        st = lax.fori_loop(0, K, row_body, init)
        for kk in range(16):
            sl = pl.ds(kk * 16, 16)
            st_v[sl] = st_v[sl] + st[kk]
        pltpu.async_copy(a_v, y_hbm.at[pl.ds(base, K)], semy)

    fire_idx(0, bufs[0])
    wait_idx(0, bufs[0])
    fire_gather(0, bufs[0])

    @pl.when(1 < nchunks)
    def _():
        fire_idx(1, bufs[1])

    def pair_body(p, carry):
        process(2 * p, 0)

        @pl.when(2 * p + 1 < nchunks)
        def _():
            process(2 * p + 1, 1)

        return carry

    lax.fori_loop(0, (nchunks + 1) // 2, pair_body, 0)
    last = (nchunks - 1) % 2

    @pl.when(last == 0)
    def _():
        pltpu.make_async_copy(y_hbm.at[pl.ds(0, K)], a_v0, semy0).wait()

    @pl.when(last == 1)
    def _():
        pltpu.make_async_copy(y_hbm.at[pl.ds(0, K)], a_v1, semy1).wait()

    pltpu.sync_copy(st_v, st_hbm.at[pl.ds(w * 2 * H, 2 * H)])


def _sc_p1(A, C, G, src, dst):
    mesh = plsc.VectorSubcoreMesh(core_axis_name="c", subcore_axis_name="s")
    buf_set = [
        pltpu.VMEM((K,), jnp.int32),
        pltpu.VMEM((K,), jnp.int32),
        pltpu.VMEM((K, H), _f32),
        pltpu.VMEM((K, H), _f32),
        pltpu.VMEM((K, H), _f32),
    ]
    f = pl.kernel(
        _sc_p1_body,
        out_type=(jax.ShapeDtypeStruct((E, H), _f32),
                  jax.ShapeDtypeStruct((NW * 2 * H,), _f32)),
        mesh=mesh,
        scratch_types=buf_set + buf_set + [
            pltpu.VMEM((2 * H,), _f32),
            pltpu.SemaphoreType.DMA,
            pltpu.SemaphoreType.DMA,
            pltpu.SemaphoreType.DMA,
            pltpu.SemaphoreType.DMA,
            pltpu.SemaphoreType.DMA,
            pltpu.SemaphoreType.DMA,
        ],
    )
    return f(A, C, G, src, dst)


def _sc_p3_body(cond_hbm, d_hbm, src_hbm, dst_hbm, out_hbm,
                idx_s0, idx_d0, m_v0, d_v0,
                idx_s1, idx_d1, m_v1, d_v1,
                acc, semg0, semg1, semsc0, semsc1, semi0, semi1):
    cid = lax.axis_index("c")
    sid = lax.axis_index("s")
    w = cid * NS + sid
    bufs = ((idx_s0, idx_d0, m_v0, d_v0, semg0, semsc0, semi0),
            (idx_s1, idx_d1, m_v1, d_v1, semg1, semsc1, semi1))

    def zrow(i, _):
        for k in range(8):
            m_v0[i, pl.ds(k * 16, 16)] = jnp.zeros((16,), _f32)
        return 0

    lax.fori_loop(0, K3, zrow, 0)
    base_r = sid * ROWS_PER_TILE
    for t in range(ROWS_PER_TILE // K3):
        pltpu.sync_copy(m_v0, acc.at[pl.ds(base_r + t * K3, K3)])
    plsc.subcore_barrier()

    nchunks = (NCHUNK3 - w + NW - 1) // NW

    def cbase(k):
        return (w + k * NW) * K3

    def fire_idx_s(k, buf):
        base = cbase(k)
        pltpu.async_copy(src_hbm.at[pl.ds(base, K3)], buf[0], buf[6])

    def wait_idx_s(k, buf):
        base = cbase(k)
        pltpu.make_async_copy(src_hbm.at[pl.ds(base, K3)], buf[0],
                              buf[6]).wait()

    def fire_gather(k, buf):
        idx_s, idx_d, m_v, d_v, semg = buf[:5]
        base = cbase(k)
        pltpu.async_copy(d_hbm.at[idx_s], d_v, semg)
        pltpu.async_copy(cond_hbm.at[pl.ds(base, K3)], m_v, semg)
        pltpu.async_copy(dst_hbm.at[pl.ds(base, K3)], idx_d, semg)

    def process(k, b):
        idx_s, idx_d, m_v, d_v, semg, semsc, _ = bufs[b]
        idx_do, m_vo = bufs[1 - b][1], bufs[1 - b][2]
        semsco = bufs[1 - b][5]

        @pl.when(k >= 1)
        def _():
            pltpu.make_async_copy(m_vo, acc.at[idx_do], semsco).wait()

        @pl.when(k + 1 < nchunks)
        def _():
            wait_idx_s(k + 1, bufs[1 - b])
            fire_gather(k + 1, bufs[1 - b])

        base = cbase(k)
        pltpu.make_async_copy(d_hbm.at[idx_s], d_v, semg).wait()
        pltpu.make_async_copy(cond_hbm.at[pl.ds(base, K3)], m_v, semg).wait()
        pltpu.make_async_copy(dst_hbm.at[pl.ds(base, K3)], idx_d, semg).wait()

        @pl.when(k + 2 < nchunks)
        def _():
            fire_idx_s(k + 2, bufs[b])

        def row_body(i, _):
            for kk in range(8):
                sl = pl.ds(kk * 16, 16)
                m_v[i, sl] = m_v[i, sl] * d_v[i, sl]
            return 0

        lax.fori_loop(0, K3, row_body, 0)
        pltpu.async_copy(m_v, acc.at[idx_d], semsc, add=True)

    fire_idx_s(0, bufs[0])
    wait_idx_s(0, bufs[0])
    fire_gather(0, bufs[0])

    @pl.when(1 < nchunks)
    def _():
        fire_idx_s(1, bufs[1])

    def pair_body(p, carry):
        process(2 * p, 0)

        @pl.when(2 * p + 1 < nchunks)
        def _():
            process(2 * p + 1, 1)

        return carry

    lax.fori_loop(0, (nchunks + 1) // 2, pair_body, 0)
    last = (nchunks - 1) % 2

    @pl.when(last == 0)
    def _():
        pltpu.make_async_copy(m_v0, acc.at[idx_d0], semsc0).wait()

    @pl.when(last == 1)
    def _():
        pltpu.make_async_copy(m_v1, acc.at[idx_d1], semsc1).wait()

    plsc.subcore_barrier()

    out_base = cid * NPAD + base_r
    for t in range(ROWS_PER_TILE // K3):
        pltpu.sync_copy(acc.at[pl.ds(base_r + t * K3, K3)], m_v0)
        pltpu.sync_copy(m_v0, out_hbm.at[pl.ds(out_base + t * K3, K3)])


def _sc_p3(cond, D, src, dst):
    mesh = plsc.VectorSubcoreMesh(core_axis_name="c", subcore_axis_name="s")
    buf_set = [
        pltpu.VMEM((K3,), jnp.int32),
        pltpu.VMEM((K3,), jnp.int32),
        pltpu.VMEM((K3, H), _f32),
        pltpu.VMEM((K3, H), _f32),
    ]
    f = pl.kernel(
        _sc_p3_body,
        out_type=jax.ShapeDtypeStruct((NC * NPAD, H), _f32),
        mesh=mesh,
        scratch_types=buf_set + buf_set + [
            pltpu.VMEM_SHARED((NPAD, H), _f32),
            pltpu.SemaphoreType.DMA,
            pltpu.SemaphoreType.DMA,
            pltpu.SemaphoreType.DMA,
            pltpu.SemaphoreType.DMA,
            pltpu.SemaphoreType.DMA,
            pltpu.SemaphoreType.DMA,
        ],
    )
    return f(cond, D, src, dst)


def _tc_pre_body(ea_ref, ea4_ref, wfbp_ref, bfb_ref, w1m_ref, b1_ref,
                 g0_ref, g1_ref, g2_ref, coef_ref):
    ea = ea_ref[...]
    centers = lax.broadcasted_iota(jnp.int32, (1, CP), 1).astype(_f32) / (STEPS - 1.0)
    g = jnp.zeros((RB, H), _f32) + bfb_ref[...]
    for a in range(7):
        col = ea[:, a:a + 1]
        dexp = jnp.exp(-((col - centers) ** 2) * STEPS)
        g = g + jnp.dot(dexp, wfbp_ref[pl.ds(a * CP, CP), :],
                        preferred_element_type=_f32)
    for l, gref in enumerate((g0_ref, g1_ref, g2_ref)):
        gref[...] = jnp.dot(g, w1m_ref[pl.ds(l * H, H), :],
                            preferred_element_type=_f32) + b1_ref[l:l + 1, :]
    coef_ref[...] = jnp.cos(1.5707963267948966 * ea4_ref[...])


def _tc_pre(edge_attr, ea4d, wfb_pad, bfb, w1m_all, b1_all):
    out_shape = (jax.ShapeDtypeStruct((E, H), _f32),) * 3 + (
        jax.ShapeDtypeStruct((NBLK, RB // H, H), _f32),)
    full = lambda shp: pl.BlockSpec(shp, lambda i: (0, 0))
    return pl.pallas_call(
        _tc_pre_body,
        grid=(NBLK,),
        in_specs=[pl.BlockSpec((RB, 7), lambda i: (i, 0)),
                  pl.BlockSpec((1, RB // H, H), lambda i: (i, 0, 0)),
                  full((7 * CP, H)), full((1, H)),
                  full((3 * H, H)), full((3, H))],
        out_specs=[pl.BlockSpec((RB, H), lambda i: (i, 0))] * 3 +
                  [pl.BlockSpec((1, RB // H, H), lambda i: (i, 0, 0))],
        out_shape=out_shape,
    )(edge_attr, ea4d, wfb_pad, bfb, w1m_all, b1_all)


def _tc_init_body(x_ref, w_ref, b_ref, wa_ref, wc_ref, wd_ref, bd_ref,
                  h_ref, a_ref, c_ref, d_ref):
    h = jnp.dot(x_ref[...], w_ref[...], preferred_element_type=_f32) + b_ref[...]
    h_ref[...] = h
    a_ref[...] = jnp.dot(h, wa_ref[...], preferred_element_type=_f32)
    c_ref[...] = jnp.dot(h, wc_ref[...], preferred_element_type=_f32)
    d_ref[...] = jnp.dot(h, wd_ref[...], preferred_element_type=_f32) + bd_ref[...]


def _tc_init(x, w, b, wa, wc, wd, bd):
    return pl.pallas_call(
        _tc_init_body,
        out_shape=(jax.ShapeDtypeStruct((N, H), _f32),) * 4,
    )(x, w, b, wa, wc, wd, bd)


def _tc_nodepre_body(h_ref, wa_ref, wc_ref, wd_ref, bd_ref, a_ref, c_ref, d_ref):
    h = h_ref[...]
    a_ref[...] = jnp.dot(h, wa_ref[...], preferred_element_type=_f32)
    c_ref[...] = jnp.dot(h, wc_ref[...], preferred_element_type=_f32)
    d_ref[...] = jnp.dot(h, wd_ref[...], preferred_element_type=_f32) + bd_ref[...]


def _tc_nodepre(h, wa, wc, wd, bd):
    return pl.pallas_call(
        _tc_nodepre_body,
        out_shape=(jax.ShapeDtypeStruct((N, H), _f32),) * 3,
    )(h, wa, wc, wd, bd)


def _bn_scale_shift(st_row_sum, st_row_sq, g, b):
    m = st_row_sum / E_F
    var = st_row_sq / E_F - m * m
    sc = g * lax.rsqrt(var + EPS)
    sh = b - m * sc
    return sc, sh


def _tc_p2_body(st_ref, bng_ref, bnb_ref, w2_ref, b2_ref, y1_ref,
                y2_ref, st2_ref):
    i = pl.program_id(0)
    straw = jnp.sum(st_ref[...], axis=0, keepdims=True)
    sc, sh = _bn_scale_shift(straw[:, 0:H], straw[:, H:2 * H],
                             bng_ref[...], bnb_ref[...])
    y1n = y1_ref[...] * sc + sh
    u = jnp.dot(y1n, w2_ref[...], preferred_element_type=_f32) + b2_ref[...]
    y2 = jnp.maximum(u, u * SLOPE)
    y2_ref[...] = y2

    @pl.when(i == 0)
    def _():
        st2_ref[...] = jnp.zeros_like(st2_ref)

    st2_ref[0:1, :] += jnp.sum(y2, axis=0, keepdims=True)
    st2_ref[1:2, :] += jnp.sum(y2 * y2, axis=0, keepdims=True)


def _tc_p2(st1, bng, bnb, w2, b2, y1):
    full = lambda shp: pl.BlockSpec(shp, lambda i: (0, 0))
    return pl.pallas_call(
        _tc_p2_body,
        grid=(NBLK,),
        in_specs=[full((NW, 2 * H)), full((1, H)), full((1, H)),
                  full((H, H)), full((1, H)),
                  pl.BlockSpec((RB, H), lambda i: (i, 0))],
        out_specs=[pl.BlockSpec((RB, H), lambda i: (i, 0)),
                   pl.BlockSpec((8, H), lambda i: (0, 0))],
        out_shape=(jax.ShapeDtypeStruct((E, H), _f32),
                   jax.ShapeDtypeStruct((8, H), _f32)),
    )(st1, bng, bnb, w2, b2, y1)


def _tc_p3a_body(st2_ref, bng_ref, bnb_ref, w3_ref, b3_ref, y2_ref, coef_ref,
                 cond_ref):
    sc, sh = _bn_scale_shift(st2_ref[0:1, :], st2_ref[1:2, :],
                             bng_ref[...], bnb_ref[...])
    y2n = y2_ref[...] * sc + sh
    fe = jnp.dot(y2n, w3_ref[...], preferred_element_type=_f32) + b3_ref[...]
    cond_ref[...] = fe * coef_ref[...]


def _tc_p3a(st2, bng, bnb, w3, b3, y2, coef):
    full = lambda shp: pl.BlockSpec(shp, lambda i: (0, 0))
    return pl.pallas_call(
        _tc_p3a_body,
        grid=(NBLK,),
        in_specs=[full((8, H)), full((1, H)), full((1, H)),
                  full((H, H)), full((1, H)),
                  pl.BlockSpec((RB, H), lambda i: (i, 0)),
                  pl.BlockSpec((RB, 1), lambda i: (i, 0))],
        out_specs=pl.BlockSpec((RB, H), lambda i: (i, 0)),
        out_shape=jax.ShapeDtypeStruct((E, H), _f32),
    )(st2, bng, bnb, w3, b3, y2, coef)


def _node_update(p0_ref, p1_ref, d_ref, h_ref, v_ref, w0_ref, b0_ref,
                 g_ref, bb_ref, w2_ref, b2_ref):
    z = (v_ref[...] * d_ref[...] + p0_ref[0, :N, :] + p1_ref[0, :N, :])
    z1 = jnp.dot(z, w0_ref[...], preferred_element_type=_f32) + b0_ref[...]
    z1 = _leaky(z1)
    m = jnp.mean(z1, axis=0, keepdims=True)
    cz = z1 - m
    var = jnp.mean(cz * cz, axis=0, keepdims=True)
    z1n = cz * lax.rsqrt(var + EPS) * g_ref[...] + bb_ref[...]
    z2 = jnp.dot(z1n, w2_ref[...], preferred_element_type=_f32) + b2_ref[...]
    return z2 + h_ref[...]


def _tc_node_body(p0_ref, p1_ref, d_ref, h_ref, v_ref, w0_ref, b0_ref,
                  g_ref, bb_ref, w2_ref, b2_ref, ho_ref):
    ho_ref[...] = _node_update(p0_ref, p1_ref, d_ref, h_ref, v_ref, w0_ref,
                               b0_ref, g_ref, bb_ref, w2_ref, b2_ref)


def _tc_nodef_body(p0_ref, p1_ref, d_ref, h_ref, v_ref, w0_ref, b0_ref,
                   g_ref, bb_ref, w2_ref, b2_ref,
                   wa_ref, wc_ref, wd_ref, bd_ref,
                   ho_ref, a_ref, c_ref, dn_ref):
    hn = _node_update(p0_ref, p1_ref, d_ref, h_ref, v_ref, w0_ref,
                      b0_ref, g_ref, bb_ref, w2_ref, b2_ref)
    ho_ref[...] = hn
    a_ref[...] = jnp.dot(hn, wa_ref[...], preferred_element_type=_f32)
    c_ref[...] = jnp.dot(hn, wc_ref[...], preferred_element_type=_f32)
    dn_ref[...] = jnp.dot(hn, wd_ref[...], preferred_element_type=_f32) + bd_ref[...]


def _fs(shp):
    return pl.BlockSpec(shp, lambda i: tuple(0 for _ in shp))


def _parts_specs():
    return [pl.BlockSpec((1, NPAD, H), lambda i: (0, 0, 0)),
            pl.BlockSpec((1, NPAD, H), lambda i: (1, 0, 0))]


_NODE_TAIL = [(N, H), (N, H), (1, H), (H, H), (1, H), (1, H), (1, H),
              (H, H), (1, H)]
_PROJ_TAIL = [(H, H), (H, H), (H, H), (1, H)]


def _tc_node(parts, D, h, v, w0, b0, g, bb, w2, b2):
    return pl.pallas_call(
        _tc_node_body,
        grid=(1,),
        in_specs=_parts_specs() + [_fs(s) for s in _NODE_TAIL],
        out_specs=_fs((N, H)),
        out_shape=jax.ShapeDtypeStruct((N, H), _f32),
    )(parts, parts, D, h, v, w0, b0, g, bb, w2, b2)


def _tc_nodef(parts, D, h, v, w0, b0, g, bb, w2, b2, wa, wc, wd, bd):
    return pl.pallas_call(
        _tc_nodef_body,
        grid=(1,),
        in_specs=_parts_specs() + [_fs(s) for s in _NODE_TAIL + _PROJ_TAIL],
        out_specs=[_fs((N, H))] * 4,
        out_shape=(jax.ShapeDtypeStruct((N, H), _f32),) * 4,
    )(parts, parts, D, h, v, w0, b0, g, bb, w2, b2, wa, wc, wd, bd)


def _tc_head_body(h_ref, w0_ref, b0_ref, g_ref, bb_ref, w1_ref, b1_ref, y_ref,
                  loss_ref, pred_ref):
    t = jnp.dot(h_ref[...], w0_ref[...], preferred_element_type=_f32) + b0_ref[...]
    t = _leaky(t)
    m = jnp.mean(t, axis=0, keepdims=True)
    ct = t - m
    var = jnp.mean(ct * ct, axis=0, keepdims=True)
    tn = ct * lax.rsqrt(var + EPS) * g_ref[...] + bb_ref[...]
    pred = jnp.dot(tn, w1_ref[...], preferred_element_type=_f32) + b1_ref[...]
    pred_ref[...] = pred
    r = pred - y_ref[...]
    loss_ref[...] = jnp.mean(r * r).reshape(1, 1)


def _tc_head(h, w0, b0, g, bb, w1, b1, y):
    return pl.pallas_call(
        _tc_head_body,
        out_shape=(jax.ShapeDtypeStruct((1, 1), _f32),
                   jax.ShapeDtypeStruct((N, 1), _f32)),
    )(h, w0, b0, g, bb, w1, b1, y)


def _row(v):
    return v.reshape(1, -1).astype(_f32)


def kernel(x, edge_attr, edge_index, y, params):
    x = x.astype(_f32)
    edge_attr = edge_attr.astype(_f32)
    y = y.astype(_f32)
    src = edge_index[0].astype(jnp.int32)
    dst = edge_index[1].astype(jnp.int32)

    p = params
    wfb = p["f_b"]["W"].astype(_f32)
    wfb_pad = jnp.zeros((7, CP, H), _f32).at[:, :STEPS, :].set(
        wfb.reshape(7, STEPS, H)).reshape(7 * CP, H)
    w1m_all = jnp.concatenate(
        [p["layers"][l]["f_e"][0]["W"][H:2 * H, :] for l in range(3)], axis=0)
    b1_all = jnp.stack([p["layers"][l]["f_e"][0]["b"] for l in range(3)], axis=0)

    ea4d = edge_attr[:, 3].reshape(NBLK, RB // H, H)
    g0, g1, g2, coef2 = _tc_pre(edge_attr, ea4d, wfb_pad, _row(p["f_b"]["b"]),
                                w1m_all.astype(_f32), b1_all.astype(_f32))
    coef = coef2.reshape(E, 1)
    gmids = (g0, g1, g2)

    def proj_w(l):
        lp = p["layers"][l]
        w1 = lp["f_e"][0]["W"].astype(_f32)
        return (w1[:H, :], w1[2 * H:, :],
                lp["f_d"]["W"].astype(_f32), _row(lp["f_d"]["b"]))

    h, A, C, D = _tc_init(x, p["f_x"]["W"].astype(_f32), _row(p["f_x"]["b"]),
                          *proj_w(0))

    for l in range(3):
        lp = p["layers"][l]
        y1, st1p = _sc_p1(A, C, gmids[l], src, dst)
        y2, st2 = _tc_p2(st1p.reshape(NW, 2 * H),
                         _row(lp["f_e"][1]["g"]), _row(lp["f_e"][1]["b"]),
                         lp["f_e"][2]["W"].astype(_f32), _row(lp["f_e"][2]["b"]),
                         y1)
        cond = _tc_p3a(st2, _row(lp["f_e"][3]["g"]), _row(lp["f_e"][3]["b"]),
                       lp["f_e"][4]["W"].astype(_f32), _row(lp["f_e"][4]["b"]),
                       y2, coef)
        parts = _sc_p3(cond, D, src, dst).reshape(2, NPAD, H)
        node_args = (parts, D, h, lp["v"].astype(_f32),
                     lp["f_n"][0]["W"].astype(_f32), _row(lp["f_n"][0]["b"]),
                     _row(lp["f_n"][1]["g"]), _row(lp["f_n"][1]["b"]),
                     lp["f_n"][2]["W"].astype(_f32), _row(lp["f_n"][2]["b"]))
        if l < 2:
            h, A, C, D = _tc_nodef(*node_args, *proj_w(l + 1))
        else:
            h = _tc_node(*node_args)

    ft = p["f_target"]
    loss, pred = _tc_head(h, ft[0]["W"].astype(_f32), _row(ft[0]["b"]),
                          _row(ft[1]["g"]), _row(ft[1]["b"]),
                          ft[2]["W"].astype(_f32), _row(ft[2]["b"]), y)
    return loss[0, 0], pred


# speedup vs baseline: 1.3591x; 1.0507x over previous
import functools

import jax
import jax.numpy as jnp
from jax import lax
from jax.experimental import pallas as pl
from jax.experimental.pallas import tpu as pltpu
from jax.experimental.pallas import tpu_sc as plsc

N = 10000
E = 160000
H = 128
STEPS = 50
CP = 64
EPS = 1e-5
SLOPE = 0.01
E_F = float(E)

RB = 6400
NBLK = E // RB
K = 128
NCHUNK = E // K
K3 = 80
NCHUNK3 = E // K3
NC, NS = 2, 16
NW = NC * NS
NPAD = 10240
ROWS_PER_TILE = NPAD // NS

_f32 = jnp.float32


def _leaky(x):
    return jnp.maximum(x, x * SLOPE)


def _sc_p1_body(a_hbm, c_hbm, g_hbm, src_hbm, dst_hbm, y_hbm, st_hbm,
                idx_s0, idx_d0, a_v0, c_v0, g_v0,
                idx_s1, idx_d1, a_v1, c_v1, g_v1,
                st_v, semg0, semg1, semy0, semy1, semi0, semi1):
    cid = lax.axis_index("c")
    sid = lax.axis_index("s")
    w = cid * NS + sid
    nchunks = (NCHUNK - w + NW - 1) // NW
    bufs = ((idx_s0, idx_d0, a_v0, c_v0, g_v0, semg0, semy0, semi0),
            (idx_s1, idx_d1, a_v1, c_v1, g_v1, semg1, semy1, semi1))
    zero = jnp.zeros((16,), _f32)
    init = (zero,) * 16
    for k in range(16):
        st_v[pl.ds(k * 16, 16)] = zero

    def cbase(k):
        return (w + k * NW) * K

    def fire_idx(k, buf):
        idx_s, idx_d = buf[0], buf[1]
        semi = buf[7]
        base = cbase(k)
        pltpu.async_copy(src_hbm.at[pl.ds(base, K)], idx_s, semi)
        pltpu.async_copy(dst_hbm.at[pl.ds(base, K)], idx_d, semi)

    def wait_idx(k, buf):
        idx_s, idx_d = buf[0], buf[1]
        semi = buf[7]
        base = cbase(k)
        pltpu.make_async_copy(src_hbm.at[pl.ds(base, K)], idx_s, semi).wait()
        pltpu.make_async_copy(dst_hbm.at[pl.ds(base, K)], idx_d, semi).wait()

    def fire_gather(k, buf):
        idx_s, idx_d, a_v, c_v, g_v, semg = buf[:6]
        base = cbase(k)
        pltpu.async_copy(a_hbm.at[idx_s], a_v, semg)
        pltpu.async_copy(c_hbm.at[idx_d], c_v, semg)
        pltpu.async_copy(g_hbm.at[pl.ds(base, K)], g_v, semg)

    def process(k, b):
        idx_s, idx_d, a_v, c_v, g_v, semg, semy, _ = bufs[b]
        a_vo = bufs[1 - b][2]
        semyo = bufs[1 - b][6]

        @pl.when(k >= 1)
        def _():
            pltpu.make_async_copy(y_hbm.at[pl.ds(0, K)], a_vo, semyo).wait()

        @pl.when(k + 1 < nchunks)
        def _():
            wait_idx(k + 1, bufs[1 - b])
            fire_gather(k + 1, bufs[1 - b])

        base = cbase(k)
        pltpu.make_async_copy(a_hbm.at[idx_s], a_v, semg).wait()
        pltpu.make_async_copy(c_hbm.at[idx_d], c_v, semg).wait()
        pltpu.make_async_copy(g_hbm.at[pl.ds(base, K)], g_v, semg).wait()

        @pl.when(k + 2 < nchunks)
        def _():
            fire_idx(k + 2, bufs[b])

        def row_body(i, st_in):
            out = list(st_in)
            for kk in range(8):
                sl = pl.ds(kk * 16, 16)
                p = a_v[i, sl] + c_v[i, sl] + g_v[i, sl]
                yv = jnp.maximum(p, p * SLOPE)
                a_v[i, sl] = yv
                out[kk] = out[kk] + yv
                out[8 + kk] = out[8 + kk] + yv * yv
            return tuple(out)

        st = lax.fori_loop(0, K, row_body, init)
        for kk in range(16):
            sl = pl.ds(kk * 16, 16)
            st_v[sl] = st_v[sl] + st[kk]
        pltpu.async_copy(a_v, y_hbm.at[pl.ds(base, K)], semy)

    fire_idx(0, bufs[0])
    wait_idx(0, bufs[0])
    fire_gather(0, bufs[0])

    @pl.when(1 < nchunks)
    def _():
        fire_idx(1, bufs[1])

    def pair_body(p, carry):
        process(2 * p, 0)

        @pl.when(2 * p + 1 < nchunks)
        def _():
            process(2 * p + 1, 1)

        return carry

    lax.fori_loop(0, (nchunks + 1) // 2, pair_body, 0)
    last = (nchunks - 1) % 2

    @pl.when(last == 0)
    def _():
        pltpu.make_async_copy(y_hbm.at[pl.ds(0, K)], a_v0, semy0).wait()

    @pl.when(last == 1)
    def _():
        pltpu.make_async_copy(y_hbm.at[pl.ds(0, K)], a_v1, semy1).wait()

    pltpu.sync_copy(st_v, st_hbm.at[pl.ds(w * 2 * H, 2 * H)])


def _sc_p1(A, C, G, src, dst):
    mesh = plsc.VectorSubcoreMesh(core_axis_name="c", subcore_axis_name="s")
    buf_set = [
        pltpu.VMEM((K,), jnp.int32),
        pltpu.VMEM((K,), jnp.int32),
        pltpu.VMEM((K, H), _f32),
        pltpu.VMEM((K, H), _f32),
        pltpu.VMEM((K, H), _f32),
    ]
    f = pl.kernel(
        _sc_p1_body,
        out_type=(jax.ShapeDtypeStruct((E, H), _f32),
                  jax.ShapeDtypeStruct((NW * 2 * H,), _f32)),
        mesh=mesh,
        scratch_types=buf_set + buf_set + [
            pltpu.VMEM((2 * H,), _f32),
            pltpu.SemaphoreType.DMA,
            pltpu.SemaphoreType.DMA,
            pltpu.SemaphoreType.DMA,
            pltpu.SemaphoreType.DMA,
            pltpu.SemaphoreType.DMA,
            pltpu.SemaphoreType.DMA,
        ],
    )
    return f(A, C, G, src, dst)


def _sc_p3_body(cond_hbm, d_hbm, src_hbm, dst_hbm, out_hbm,
                idx_s0, idx_d0, m_v0, d_v0,
                idx_s1, idx_d1, m_v1, d_v1,
                acc, semg0, semg1, semsc0, semsc1, semi0, semi1):
    cid = lax.axis_index("c")
    sid = lax.axis_index("s")
    w = cid * NS + sid
    bufs = ((idx_s0, idx_d0, m_v0, d_v0, semg0, semsc0, semi0),
            (idx_s1, idx_d1, m_v1, d_v1, semg1, semsc1, semi1))

    def zrow(i, _):
        for k in range(8):
            m_v0[i, pl.ds(k * 16, 16)] = jnp.zeros((16,), _f32)
        return 0

    lax.fori_loop(0, K3, zrow, 0)
    base_r = sid * ROWS_PER_TILE
    for t in range(ROWS_PER_TILE // K3):
        pltpu.sync_copy(m_v0, acc.at[pl.ds(base_r + t * K3, K3)])
    plsc.subcore_barrier()

    nchunks = (NCHUNK3 - w + NW - 1) // NW

    def cbase(k):
        return (w + k * NW) * K3

    def fire_idx_s(k, buf):
        base = cbase(k)
        pltpu.async_copy(src_hbm.at[pl.ds(base, K3)], buf[0], buf[6])

    def wait_idx_s(k, buf):
        base = cbase(k)
        pltpu.make_async_copy(src_hbm.at[pl.ds(base, K3)], buf[0],
                              buf[6]).wait()

    def fire_gather(k, buf):
        idx_s, idx_d, m_v, d_v, semg = buf[:5]
        base = cbase(k)
        pltpu.async_copy(d_hbm.at[idx_s], d_v, semg)
        pltpu.async_copy(cond_hbm.at[pl.ds(base, K3)], m_v, semg)
        pltpu.async_copy(dst_hbm.at[pl.ds(base, K3)], idx_d, semg)

    def process(k, b):
        idx_s, idx_d, m_v, d_v, semg, semsc, _ = bufs[b]
        idx_do, m_vo = bufs[1 - b][1], bufs[1 - b][2]
        semsco = bufs[1 - b][5]

        @pl.when(k >= 1)
        def _():
            pltpu.make_async_copy(m_vo, acc.at[idx_do], semsco).wait()

        @pl.when(k + 1 < nchunks)
        def _():
            wait_idx_s(k + 1, bufs[1 - b])
            fire_gather(k + 1, bufs[1 - b])

        base = cbase(k)
        pltpu.make_async_copy(d_hbm.at[idx_s], d_v, semg).wait()
        pltpu.make_async_copy(cond_hbm.at[pl.ds(base, K3)], m_v, semg).wait()
        pltpu.make_async_copy(dst_hbm.at[pl.ds(base, K3)], idx_d, semg).wait()

        @pl.when(k + 2 < nchunks)
        def _():
            fire_idx_s(k + 2, bufs[b])

        def row_body(i, _):
            for kk in range(8):
                sl = pl.ds(kk * 16, 16)
                m_v[i, sl] = m_v[i, sl] * d_v[i, sl]
            return 0

        lax.fori_loop(0, K3, row_body, 0)
        pltpu.async_copy(m_v, acc.at[idx_d], semsc, add=True)

    fire_idx_s(0, bufs[0])
    wait_idx_s(0, bufs[0])
    fire_gather(0, bufs[0])

    @pl.when(1 < nchunks)
    def _():
        fire_idx_s(1, bufs[1])

    def pair_body(p, carry):
        process(2 * p, 0)

        @pl.when(2 * p + 1 < nchunks)
        def _():
            process(2 * p + 1, 1)

        return carry

    lax.fori_loop(0, (nchunks + 1) // 2, pair_body, 0)
    last = (nchunks - 1) % 2

    @pl.when(last == 0)
    def _():
        pltpu.make_async_copy(m_v0, acc.at[idx_d0], semsc0).wait()

    @pl.when(last == 1)
    def _():
        pltpu.make_async_copy(m_v1, acc.at[idx_d1], semsc1).wait()

    plsc.subcore_barrier()

    out_base = cid * NPAD + base_r
    for t in range(ROWS_PER_TILE // K3):
        pltpu.sync_copy(acc.at[pl.ds(base_r + t * K3, K3)], m_v0)
        pltpu.sync_copy(m_v0, out_hbm.at[pl.ds(out_base + t * K3, K3)])


def _sc_p3(cond, D, src, dst):
    mesh = plsc.VectorSubcoreMesh(core_axis_name="c", subcore_axis_name="s")
    buf_set = [
        pltpu.VMEM((K3,), jnp.int32),
        pltpu.VMEM((K3,), jnp.int32),
        pltpu.VMEM((K3, H), _f32),
        pltpu.VMEM((K3, H), _f32),
    ]
    f = pl.kernel(
        _sc_p3_body,
        out_type=jax.ShapeDtypeStruct((NC * NPAD, H), _f32),
        mesh=mesh,
        scratch_types=buf_set + buf_set + [
            pltpu.VMEM_SHARED((NPAD, H), _f32),
            pltpu.SemaphoreType.DMA,
            pltpu.SemaphoreType.DMA,
            pltpu.SemaphoreType.DMA,
            pltpu.SemaphoreType.DMA,
            pltpu.SemaphoreType.DMA,
            pltpu.SemaphoreType.DMA,
        ],
    )
    return f(cond, D, src, dst)


def _tc_pre_body(ea_ref, ea4_ref, wfbp_ref, bfb_ref, w1m_ref, b1_ref,
                 g0_ref, g1_ref, g2_ref, coef_ref):
    ea = ea_ref[...]
    centers = lax.broadcasted_iota(jnp.int32, (1, CP), 1).astype(_f32) / (STEPS - 1.0)
    g = jnp.zeros((RB, H), _f32) + bfb_ref[...]
    for a in range(7):
        col = ea[:, a:a + 1]
        dexp = jnp.exp(-((col - centers) ** 2) * STEPS)
        g = g + jnp.dot(dexp, wfbp_ref[pl.ds(a * CP, CP), :],
                        preferred_element_type=_f32)
    for l, gref in enumerate((g0_ref, g1_ref, g2_ref)):
        gref[...] = jnp.dot(g, w1m_ref[pl.ds(l * H, H), :],
                            preferred_element_type=_f32) + b1_ref[l:l + 1, :]
    coef_ref[...] = jnp.cos(1.5707963267948966 * ea4_ref[...])


def _tc_pre(edge_attr, ea4d, wfb_pad, bfb, w1m_all, b1_all):
    out_shape = (jax.ShapeDtypeStruct((E, H), _f32),) * 3 + (
        jax.ShapeDtypeStruct((NBLK, RB // H, H), _f32),)
    full = lambda shp: pl.BlockSpec(shp, lambda i: (0, 0))
    return pl.pallas_call(
        _tc_pre_body,
        grid=(NBLK,),
        in_specs=[pl.BlockSpec((RB, 7), lambda i: (i, 0)),
                  pl.BlockSpec((1, RB // H, H), lambda i: (i, 0, 0)),
                  full((7 * CP, H)), full((1, H)),
                  full((3 * H, H)), full((3, H))],
        out_specs=[pl.BlockSpec((RB, H), lambda i: (i, 0))] * 3 +
                  [pl.BlockSpec((1, RB // H, H), lambda i: (i, 0, 0))],
        out_shape=out_shape,
    )(edge_attr, ea4d, wfb_pad, bfb, w1m_all, b1_all)


def _tc_init_body(x_ref, w_ref, b_ref, wa_ref, wc_ref, wd_ref, bd_ref,
                  h_ref, a_ref, c_ref, d_ref):
    h = jnp.dot(x_ref[...], w_ref[...], preferred_element_type=_f32) + b_ref[...]
    h_ref[...] = h
    a_ref[...] = jnp.dot(h, wa_ref[...], preferred_element_type=_f32)
    c_ref[...] = jnp.dot(h, wc_ref[...], preferred_element_type=_f32)
    d_ref[...] = jnp.dot(h, wd_ref[...], preferred_element_type=_f32) + bd_ref[...]


def _tc_init(x, w, b, wa, wc, wd, bd):
    return pl.pallas_call(
        _tc_init_body,
        out_shape=(jax.ShapeDtypeStruct((N, H), _f32),) * 4,
    )(x, w, b, wa, wc, wd, bd)


def _tc_nodepre_body(h_ref, wa_ref, wc_ref, wd_ref, bd_ref, a_ref, c_ref, d_ref):
    h = h_ref[...]
    a_ref[...] = jnp.dot(h, wa_ref[...], preferred_element_type=_f32)
    c_ref[...] = jnp.dot(h, wc_ref[...], preferred_element_type=_f32)
    d_ref[...] = jnp.dot(h, wd_ref[...], preferred_element_type=_f32) + bd_ref[...]


def _tc_nodepre(h, wa, wc, wd, bd):
    return pl.pallas_call(
        _tc_nodepre_body,
        out_shape=(jax.ShapeDtypeStruct((N, H), _f32),) * 3,
    )(h, wa, wc, wd, bd)


def _bn_scale_shift(st_row_sum, st_row_sq, g, b):
    m = st_row_sum / E_F
    var = st_row_sq / E_F - m * m
    sc = g * lax.rsqrt(var + EPS)
    sh = b - m * sc
    return sc, sh


def _tc_p2_body(st_ref, bng_ref, bnb_ref, w2_ref, b2_ref, y1_ref,
                y2_ref, st2_ref):
    i = pl.program_id(0)
    straw = jnp.sum(st_ref[...], axis=0, keepdims=True)
    sc, sh = _bn_scale_shift(straw[:, 0:H], straw[:, H:2 * H],
                             bng_ref[...], bnb_ref[...])
    y1n = y1_ref[...] * sc + sh
    u = jnp.dot(y1n, w2_ref[...], preferred_element_type=_f32) + b2_ref[...]
    y2 = jnp.maximum(u, u * SLOPE)
    y2_ref[...] = y2

    @pl.when(i == 0)
    def _():
        st2_ref[...] = jnp.zeros_like(st2_ref)

    st2_ref[0:1, :] += jnp.sum(y2, axis=0, keepdims=True)
    st2_ref[1:2, :] += jnp.sum(y2 * y2, axis=0, keepdims=True)


def _tc_p2(st1, bng, bnb, w2, b2, y1):
    full = lambda shp: pl.BlockSpec(shp, lambda i: (0, 0))
    return pl.pallas_call(
        _tc_p2_body,
        grid=(NBLK,),
        in_specs=[full((NW, 2 * H)), full((1, H)), full((1, H)),
                  full((H, H)), full((1, H)),
                  pl.BlockSpec((RB, H), lambda i: (i, 0))],
        out_specs=[pl.BlockSpec((RB, H), lambda i: (i, 0)),
                   pl.BlockSpec((8, H), lambda i: (0, 0))],
        out_shape=(jax.ShapeDtypeStruct((E, H), _f32),
                   jax.ShapeDtypeStruct((8, H), _f32)),
    )(st1, bng, bnb, w2, b2, y1)


def _tc_p3a_body(st2_ref, bng_ref, bnb_ref, w3_ref, b3_ref, y2_ref, coef_ref,
                 cond_ref):
    sc, sh = _bn_scale_shift(st2_ref[0:1, :], st2_ref[1:2, :],
                             bng_ref[...], bnb_ref[...])
    y2n = y2_ref[...] * sc + sh
    fe = jnp.dot(y2n, w3_ref[...], preferred_element_type=_f32) + b3_ref[...]
    cond_ref[...] = fe * coef_ref[...]


def _tc_p3a(st2, bng, bnb, w3, b3, y2, coef):
    full = lambda shp: pl.BlockSpec(shp, lambda i: (0, 0))
    return pl.pallas_call(
        _tc_p3a_body,
        grid=(NBLK,),
        in_specs=[full((8, H)), full((1, H)), full((1, H)),
                  full((H, H)), full((1, H)),
                  pl.BlockSpec((RB, H), lambda i: (i, 0)),
                  pl.BlockSpec((RB, 1), lambda i: (i, 0))],
        out_specs=pl.BlockSpec((RB, H), lambda i: (i, 0)),
        out_shape=jax.ShapeDtypeStruct((E, H), _f32),
    )(st2, bng, bnb, w3, b3, y2, coef)


def _node_update(p0_ref, p1_ref, d_ref, h_ref, v_ref, w0_ref, b0_ref,
                 g_ref, bb_ref, w2_ref, b2_ref):
    z = (v_ref[...] * d_ref[...] + p0_ref[0, :N, :] + p1_ref[0, :N, :])
    z1 = jnp.dot(z, w0_ref[...], preferred_element_type=_f32) + b0_ref[...]
    z1 = _leaky(z1)
    m = jnp.mean(z1, axis=0, keepdims=True)
    cz = z1 - m
    var = jnp.mean(cz * cz, axis=0, keepdims=True)
    z1n = cz * lax.rsqrt(var + EPS) * g_ref[...] + bb_ref[...]
    z2 = jnp.dot(z1n, w2_ref[...], preferred_element_type=_f32) + b2_ref[...]
    return z2 + h_ref[...]


def _tc_node_body(p0_ref, p1_ref, d_ref, h_ref, v_ref, w0_ref, b0_ref,
                  g_ref, bb_ref, w2_ref, b2_ref, ho_ref):
    ho_ref[...] = _node_update(p0_ref, p1_ref, d_ref, h_ref, v_ref, w0_ref,
                               b0_ref, g_ref, bb_ref, w2_ref, b2_ref)


def _tc_nodef_body(p0_ref, p1_ref, d_ref, h_ref, v_ref, w0_ref, b0_ref,
                   g_ref, bb_ref, w2_ref, b2_ref,
                   wa_ref, wc_ref, wd_ref, bd_ref,
                   ho_ref, a_ref, c_ref, dn_ref):
    hn = _node_update(p0_ref, p1_ref, d_ref, h_ref, v_ref, w0_ref,
                      b0_ref, g_ref, bb_ref, w2_ref, b2_ref)
    ho_ref[...] = hn
    a_ref[...] = jnp.dot(hn, wa_ref[...], preferred_element_type=_f32)
    c_ref[...] = jnp.dot(hn, wc_ref[...], preferred_element_type=_f32)
    dn_ref[...] = jnp.dot(hn, wd_ref[...], preferred_element_type=_f32) + bd_ref[...]


def _fs(shp):
    return pl.BlockSpec(shp, lambda i: tuple(0 for _ in shp))


def _parts_specs():
    return [pl.BlockSpec((1, NPAD, H), lambda i: (0, 0, 0)),
            pl.BlockSpec((1, NPAD, H), lambda i: (1, 0, 0))]


_NODE_TAIL = [(N, H), (N, H), (1, H), (H, H), (1, H), (1, H), (1, H),
              (H, H), (1, H)]
_PROJ_TAIL = [(H, H), (H, H), (H, H), (1, H)]


def _tc_node(parts, D, h, v, w0, b0, g, bb, w2, b2):
    return pl.pallas_call(
        _tc_node_body,
        grid=(1,),
        in_specs=_parts_specs() + [_fs(s) for s in _NODE_TAIL],
        out_specs=_fs((N, H)),
        out_shape=jax.ShapeDtypeStruct((N, H), _f32),
    )(parts, parts, D, h, v, w0, b0, g, bb, w2, b2)


def _tc_nodef(parts, D, h, v, w0, b0, g, bb, w2, b2, wa, wc, wd, bd):
    return pl.pallas_call(
        _tc_nodef_body,
        grid=(1,),
        in_specs=_parts_specs() + [_fs(s) for s in _NODE_TAIL + _PROJ_TAIL],
        out_specs=[_fs((N, H))] * 4,
        out_shape=(jax.ShapeDtypeStruct((N, H), _f32),) * 4,
    )(parts, parts, D, h, v, w0, b0, g, bb, w2, b2, wa, wc, wd, bd)


def _tc_head_body(h_ref, w0_ref, b0_ref, g_ref, bb_ref, w1_ref, b1_ref, y_ref,
                  loss_ref, pred_ref):
    t = jnp.dot(h_ref[...], w0_ref[...], preferred_element_type=_f32) + b0_ref[...]
    t = _leaky(t)
    m = jnp.mean(t, axis=0, keepdims=True)
    ct = t - m
    var = jnp.mean(ct * ct, axis=0, keepdims=True)
    tn = ct * lax.rsqrt(var + EPS) * g_ref[...] + bb_ref[...]
    pred = jnp.dot(tn, w1_ref[...], preferred_element_type=_f32) + b1_ref[...]
    pred_ref[...] = pred
    r = pred - y_ref[...]
    loss_ref[...] = jnp.mean(r * r).reshape(1, 1)


def _tc_head(h, w0, b0, g, bb, w1, b1, y):
    return pl.pallas_call(
        _tc_head_body,
        out_shape=(jax.ShapeDtypeStruct((1, 1), _f32),
                   jax.ShapeDtypeStruct((N, 1), _f32)),
    )(h, w0, b0, g, bb, w1, b1, y)


def _row(v):
    return v.reshape(1, -1).astype(_f32)


def kernel(x, edge_attr, edge_index, y, params):
    x = x.astype(_f32)
    edge_attr = edge_attr.astype(_f32)
    y = y.astype(_f32)
    src = edge_index[0].astype(jnp.int32)
    dst = edge_index[1].astype(jnp.int32)

    p = params
    wfb = p["f_b"]["W"].astype(_f32)
    wfb_pad = jnp.zeros((7, CP, H), _f32).at[:, :STEPS, :].set(
        wfb.reshape(7, STEPS, H)).reshape(7 * CP, H)
    w1m_all = jnp.concatenate(
        [p["layers"][l]["f_e"][0]["W"][H:2 * H, :] for l in range(3)], axis=0)
    b1_all = jnp.stack([p["layers"][l]["f_e"][0]["b"] for l in range(3)], axis=0)

    ea4d = edge_attr[:, 3].reshape(NBLK, RB // H, H)
    g0, g1, g2, coef2 = _tc_pre(edge_attr, ea4d, wfb_pad, _row(p["f_b"]["b"]),
                                w1m_all.astype(_f32), b1_all.astype(_f32))
    coef = coef2.reshape(E, 1)
    gmids = (g0, g1, g2)

    def proj_w(l):
        lp = p["layers"][l]
        w1 = lp["f_e"][0]["W"].astype(_f32)
        return (w1[:H, :], w1[2 * H:, :],
                lp["f_d"]["W"].astype(_f32), _row(lp["f_d"]["b"]))

    h, A, C, D = _tc_init(x, p["f_x"]["W"].astype(_f32), _row(p["f_x"]["b"]),
                          *proj_w(0))

    for l in range(3):
        lp = p["layers"][l]
        y1, st1p = _sc_p1(A, C, gmids[l], src, dst)
        y2, st2 = _tc_p2(st1p.reshape(NW, 2 * H),
                         _row(lp["f_e"][1]["g"]), _row(lp["f_e"][1]["b"]),
                         lp["f_e"][2]["W"].astype(_f32), _row(lp["f_e"][2]["b"]),
                         y1)
        cond = _tc_p3a(st2, _row(lp["f_e"][3]["g"]), _row(lp["f_e"][3]["b"]),
                       lp["f_e"][4]["W"].astype(_f32), _row(lp["f_e"][4]["b"]),
                       y2, coef)
        parts = _sc_p3(cond, D, src, dst).reshape(2, NPAD, H)
        node_args = (parts, D, h, lp["v"].astype(_f32),
                     lp["f_n"][0]["W"].astype(_f32), _row(lp["f_n"][0]["b"]),
                     _row(lp["f_n"][1]["g"]), _row(lp["f_n"][1]["b"]),
                     lp["f_n"][2]["W"].astype(_f32), _row(lp["f_n"][2]["b"]))
        if l < 2:
            h, A, C, D = _tc_nodef(*node_args, *proj_w(l + 1))
        else:
            h = _tc_node(*node_args)

    ft = p["f_target"]
    loss, pred = _tc_head(h, ft[0]["W"].astype(_f32), _row(ft[0]["b"]),
                          _row(ft[1]["g"]), _row(ft[1]["b"]),
                          ft[2]["W"].astype(_f32), _row(ft[2]["b"]), y)
    return loss[0, 0], pred


# speedup vs baseline: 1.3761x; 1.0125x over previous
import functools

import jax
import jax.numpy as jnp
from jax import lax
from jax.experimental import pallas as pl
from jax.experimental.pallas import tpu as pltpu
from jax.experimental.pallas import tpu_sc as plsc

N = 10000
E = 160000
H = 128
STEPS = 50
CP = 64
EPS = 1e-5
SLOPE = 0.01
E_F = float(E)

RB = 6400
NBLK = E // RB
RBE = 16000
NBLKE = E // RBE
K = 128
NCHUNK = E // K
K3 = 80
NCHUNK3 = E // K3
NC, NS = 2, 16
NW = NC * NS
NPAD = 10240
ROWS_PER_TILE = NPAD // NS

_f32 = jnp.float32


def _leaky(x):
    return jnp.maximum(x, x * SLOPE)


def _sc_p1_body(a_hbm, c_hbm, g_hbm, src_hbm, dst_hbm, y_hbm, st_hbm,
                idx_s0, idx_d0, a_v0, c_v0, g_v0,
                idx_s1, idx_d1, a_v1, c_v1, g_v1,
                st_v, semg0, semg1, semy0, semy1, semi0, semi1):
    cid = lax.axis_index("c")
    sid = lax.axis_index("s")
    w = cid * NS + sid
    nchunks = (NCHUNK - w + NW - 1) // NW
    bufs = ((idx_s0, idx_d0, a_v0, c_v0, g_v0, semg0, semy0, semi0),
            (idx_s1, idx_d1, a_v1, c_v1, g_v1, semg1, semy1, semi1))
    zero = jnp.zeros((16,), _f32)
    init = (zero,) * 16
    for k in range(16):
        st_v[pl.ds(k * 16, 16)] = zero

    def cbase(k):
        return (w + k * NW) * K

    def fire_idx(k, buf):
        idx_s, idx_d = buf[0], buf[1]
        semi = buf[7]
        base = cbase(k)
        pltpu.async_copy(src_hbm.at[pl.ds(base, K)], idx_s, semi)
        pltpu.async_copy(dst_hbm.at[pl.ds(base, K)], idx_d, semi)

    def wait_idx(k, buf):
        idx_s, idx_d = buf[0], buf[1]
        semi = buf[7]
        base = cbase(k)
        pltpu.make_async_copy(src_hbm.at[pl.ds(base, K)], idx_s, semi).wait()
        pltpu.make_async_copy(dst_hbm.at[pl.ds(base, K)], idx_d, semi).wait()

    def fire_gather(k, buf):
        idx_s, idx_d, a_v, c_v, g_v, semg = buf[:6]
        base = cbase(k)
        pltpu.async_copy(a_hbm.at[idx_s], a_v, semg)
        pltpu.async_copy(c_hbm.at[idx_d], c_v, semg)
        pltpu.async_copy(g_hbm.at[pl.ds(base, K)], g_v, semg)

    def process(k, b):
        idx_s, idx_d, a_v, c_v, g_v, semg, semy, _ = bufs[b]
        a_vo = bufs[1 - b][2]
        semyo = bufs[1 - b][6]

        @pl.when(k >= 1)
        def _():
            pltpu.make_async_copy(y_hbm.at[pl.ds(0, K)], a_vo, semyo).wait()

        @pl.when(k + 1 < nchunks)
        def _():
            wait_idx(k + 1, bufs[1 - b])
            fire_gather(k + 1, bufs[1 - b])

        base = cbase(k)
        pltpu.make_async_copy(a_hbm.at[idx_s], a_v, semg).wait()
        pltpu.make_async_copy(c_hbm.at[idx_d], c_v, semg).wait()
        pltpu.make_async_copy(g_hbm.at[pl.ds(base, K)], g_v, semg).wait()

        @pl.when(k + 2 < nchunks)
        def _():
            fire_idx(k + 2, bufs[b])

        def row_body(i, st_in):
            out = list(st_in)
            for kk in range(8):
                sl = pl.ds(kk * 16, 16)
                p = a_v[i, sl] + c_v[i, sl] + g_v[i, sl]
                yv = jnp.maximum(p, p * SLOPE)
                a_v[i, sl] = yv
                out[kk] = out[kk] + yv
                out[8 + kk] = out[8 + kk] + yv * yv
            return tuple(out)

        st = lax.fori_loop(0, K, row_body, init)
        for kk in range(16):
            sl = pl.ds(kk * 16, 16)
            st_v[sl] = st_v[sl] + st[kk]
        pltpu.async_copy(a_v, y_hbm.at[pl.ds(base, K)], semy)

    fire_idx(0, bufs[0])
    wait_idx(0, bufs[0])
    fire_gather(0, bufs[0])

    @pl.when(1 < nchunks)
    def _():
        fire_idx(1, bufs[1])

    def pair_body(p, carry):
        process(2 * p, 0)

        @pl.when(2 * p + 1 < nchunks)
        def _():
            process(2 * p + 1, 1)

        return carry

    lax.fori_loop(0, (nchunks + 1) // 2, pair_body, 0)
    last = (nchunks - 1) % 2

    @pl.when(last == 0)
    def _():
        pltpu.make_async_copy(y_hbm.at[pl.ds(0, K)], a_v0, semy0).wait()

    @pl.when(last == 1)
    def _():
        pltpu.make_async_copy(y_hbm.at[pl.ds(0, K)], a_v1, semy1).wait()

    pltpu.sync_copy(st_v, st_hbm.at[pl.ds(w * 2 * H, 2 * H)])


def _sc_p1(A, C, G, src, dst):
    mesh = plsc.VectorSubcoreMesh(core_axis_name="c", subcore_axis_name="s")
    buf_set = [
        pltpu.VMEM((K,), jnp.int32),
        pltpu.VMEM((K,), jnp.int32),
        pltpu.VMEM((K, H), _f32),
        pltpu.VMEM((K, H), _f32),
        pltpu.VMEM((K, H), _f32),
    ]
    f = pl.kernel(
        _sc_p1_body,
        out_type=(jax.ShapeDtypeStruct((E, H), _f32),
                  jax.ShapeDtypeStruct((NW * 2 * H,), _f32)),
        mesh=mesh,
        scratch_types=buf_set + buf_set + [
            pltpu.VMEM((2 * H,), _f32),
            pltpu.SemaphoreType.DMA,
            pltpu.SemaphoreType.DMA,
            pltpu.SemaphoreType.DMA,
            pltpu.SemaphoreType.DMA,
            pltpu.SemaphoreType.DMA,
            pltpu.SemaphoreType.DMA,
        ],
    )
    return f(A, C, G, src, dst)


def _sc_p3_body(cond_hbm, d_hbm, src_hbm, dst_hbm, out_hbm,
                idx_s0, idx_d0, m_v0, d_v0,
                idx_s1, idx_d1, m_v1, d_v1,
                acc, semg0, semg1, semsc0, semsc1, semi0, semi1):
    cid = lax.axis_index("c")
    sid = lax.axis_index("s")
    w = cid * NS + sid
    bufs = ((idx_s0, idx_d0, m_v0, d_v0, semg0, semsc0, semi0),
            (idx_s1, idx_d1, m_v1, d_v1, semg1, semsc1, semi1))

    def zrow(i, _):
        for k in range(8):
            m_v0[i, pl.ds(k * 16, 16)] = jnp.zeros((16,), _f32)
        return 0

    lax.fori_loop(0, K3, zrow, 0)
    base_r = sid * ROWS_PER_TILE
    for t in range(ROWS_PER_TILE // K3):
        pltpu.sync_copy(m_v0, acc.at[pl.ds(base_r + t * K3, K3)])
    plsc.subcore_barrier()

    nchunks = (NCHUNK3 - w + NW - 1) // NW

    def cbase(k):
        return (w + k * NW) * K3

    def fire_idx_s(k, buf):
        base = cbase(k)
        pltpu.async_copy(src_hbm.at[pl.ds(base, K3)], buf[0], buf[6])

    def wait_idx_s(k, buf):
        base = cbase(k)
        pltpu.make_async_copy(src_hbm.at[pl.ds(base, K3)], buf[0],
                              buf[6]).wait()

    def fire_gather(k, buf):
        idx_s, idx_d, m_v, d_v, semg = buf[:5]
        base = cbase(k)
        pltpu.async_copy(d_hbm.at[idx_s], d_v, semg)
        pltpu.async_copy(cond_hbm.at[pl.ds(base, K3)], m_v, semg)
        pltpu.async_copy(dst_hbm.at[pl.ds(base, K3)], idx_d, semg)

    def process(k, b):
        idx_s, idx_d, m_v, d_v, semg, semsc, _ = bufs[b]
        idx_do, m_vo = bufs[1 - b][1], bufs[1 - b][2]
        semsco = bufs[1 - b][5]

        @pl.when(k >= 1)
        def _():
            pltpu.make_async_copy(m_vo, acc.at[idx_do], semsco).wait()

        @pl.when(k + 1 < nchunks)
        def _():
            wait_idx_s(k + 1, bufs[1 - b])
            fire_gather(k + 1, bufs[1 - b])

        base = cbase(k)
        pltpu.make_async_copy(d_hbm.at[idx_s], d_v, semg).wait()
        pltpu.make_async_copy(cond_hbm.at[pl.ds(base, K3)], m_v, semg).wait()
        pltpu.make_async_copy(dst_hbm.at[pl.ds(base, K3)], idx_d, semg).wait()

        @pl.when(k + 2 < nchunks)
        def _():
            fire_idx_s(k + 2, bufs[b])

        def row_body(i, _):
            for kk in range(8):
                sl = pl.ds(kk * 16, 16)
                m_v[i, sl] = m_v[i, sl] * d_v[i, sl]
            return 0

        lax.fori_loop(0, K3, row_body, 0)
        pltpu.async_copy(m_v, acc.at[idx_d], semsc, add=True)

    fire_idx_s(0, bufs[0])
    wait_idx_s(0, bufs[0])
    fire_gather(0, bufs[0])

    @pl.when(1 < nchunks)
    def _():
        fire_idx_s(1, bufs[1])

    def pair_body(p, carry):
        process(2 * p, 0)

        @pl.when(2 * p + 1 < nchunks)
        def _():
            process(2 * p + 1, 1)

        return carry

    lax.fori_loop(0, (nchunks + 1) // 2, pair_body, 0)
    last = (nchunks - 1) % 2

    @pl.when(last == 0)
    def _():
        pltpu.make_async_copy(m_v0, acc.at[idx_d0], semsc0).wait()

    @pl.when(last == 1)
    def _():
        pltpu.make_async_copy(m_v1, acc.at[idx_d1], semsc1).wait()

    plsc.subcore_barrier()

    out_base = cid * NPAD + base_r
    for t in range(ROWS_PER_TILE // K3):
        pltpu.sync_copy(acc.at[pl.ds(base_r + t * K3, K3)], m_v0)
        pltpu.sync_copy(m_v0, out_hbm.at[pl.ds(out_base + t * K3, K3)])


def _sc_p3(cond, D, src, dst):
    mesh = plsc.VectorSubcoreMesh(core_axis_name="c", subcore_axis_name="s")
    buf_set = [
        pltpu.VMEM((K3,), jnp.int32),
        pltpu.VMEM((K3,), jnp.int32),
        pltpu.VMEM((K3, H), _f32),
        pltpu.VMEM((K3, H), _f32),
    ]
    f = pl.kernel(
        _sc_p3_body,
        out_type=jax.ShapeDtypeStruct((NC * NPAD, H), _f32),
        mesh=mesh,
        scratch_types=buf_set + buf_set + [
            pltpu.VMEM_SHARED((NPAD, H), _f32),
            pltpu.SemaphoreType.DMA,
            pltpu.SemaphoreType.DMA,
            pltpu.SemaphoreType.DMA,
            pltpu.SemaphoreType.DMA,
            pltpu.SemaphoreType.DMA,
            pltpu.SemaphoreType.DMA,
        ],
    )
    return f(cond, D, src, dst)


def _tc_pre_body(ea_ref, ea4_ref, wfbp_ref, bfb_ref, w1m_ref, b1_ref,
                 g0_ref, g1_ref, g2_ref, coef_ref):
    ea = ea_ref[...]
    centers = lax.broadcasted_iota(jnp.int32, (1, CP), 1).astype(_f32) / (STEPS - 1.0)
    g = jnp.zeros((RB, H), _f32) + bfb_ref[...]
    for a in range(7):
        col = ea[:, a:a + 1]
        dexp = jnp.exp(-((col - centers) ** 2) * STEPS)
        g = g + jnp.dot(dexp, wfbp_ref[pl.ds(a * CP, CP), :],
                        preferred_element_type=_f32)
    for l, gref in enumerate((g0_ref, g1_ref, g2_ref)):
        gref[...] = jnp.dot(g, w1m_ref[pl.ds(l * H, H), :],
                            preferred_element_type=_f32) + b1_ref[l:l + 1, :]
    coef_ref[...] = jnp.cos(1.5707963267948966 * ea4_ref[...])


def _tc_pre(edge_attr, ea4d, wfb_pad, bfb, w1m_all, b1_all):
    out_shape = (jax.ShapeDtypeStruct((E, H), _f32),) * 3 + (
        jax.ShapeDtypeStruct((NBLK, RB // H, H), _f32),)
    full = lambda shp: pl.BlockSpec(shp, lambda i: (0, 0))
    return pl.pallas_call(
        _tc_pre_body,
        grid=(NBLK,),
        in_specs=[pl.BlockSpec((RB, 7), lambda i: (i, 0)),
                  pl.BlockSpec((1, RB // H, H), lambda i: (i, 0, 0)),
                  full((7 * CP, H)), full((1, H)),
                  full((3 * H, H)), full((3, H))],
        out_specs=[pl.BlockSpec((RB, H), lambda i: (i, 0))] * 3 +
                  [pl.BlockSpec((1, RB // H, H), lambda i: (i, 0, 0))],
        out_shape=out_shape,
    )(edge_attr, ea4d, wfb_pad, bfb, w1m_all, b1_all)


def _tc_init_body(x_ref, w_ref, b_ref, wa_ref, wc_ref, wd_ref, bd_ref,
                  h_ref, a_ref, c_ref, d_ref):
    h = jnp.dot(x_ref[...], w_ref[...], preferred_element_type=_f32) + b_ref[...]
    h_ref[...] = h
    a_ref[...] = jnp.dot(h, wa_ref[...], preferred_element_type=_f32)
    c_ref[...] = jnp.dot(h, wc_ref[...], preferred_element_type=_f32)
    d_ref[...] = jnp.dot(h, wd_ref[...], preferred_element_type=_f32) + bd_ref[...]


def _tc_init(x, w, b, wa, wc, wd, bd):
    return pl.pallas_call(
        _tc_init_body,
        out_shape=(jax.ShapeDtypeStruct((N, H), _f32),) * 4,
    )(x, w, b, wa, wc, wd, bd)


def _tc_nodepre_body(h_ref, wa_ref, wc_ref, wd_ref, bd_ref, a_ref, c_ref, d_ref):
    h = h_ref[...]
    a_ref[...] = jnp.dot(h, wa_ref[...], preferred_element_type=_f32)
    c_ref[...] = jnp.dot(h, wc_ref[...], preferred_element_type=_f32)
    d_ref[...] = jnp.dot(h, wd_ref[...], preferred_element_type=_f32) + bd_ref[...]


def _tc_nodepre(h, wa, wc, wd, bd):
    return pl.pallas_call(
        _tc_nodepre_body,
        out_shape=(jax.ShapeDtypeStruct((N, H), _f32),) * 3,
    )(h, wa, wc, wd, bd)


def _bn_scale_shift(st_row_sum, st_row_sq, g, b):
    m = st_row_sum / E_F
    var = st_row_sq / E_F - m * m
    sc = g * lax.rsqrt(var + EPS)
    sh = b - m * sc
    return sc, sh


def _tc_p2_body(st_ref, bng_ref, bnb_ref, w2_ref, b2_ref, y1_ref,
                y2_ref, st2_ref):
    i = pl.program_id(0)
    straw = jnp.sum(st_ref[...], axis=0, keepdims=True)
    sc, sh = _bn_scale_shift(straw[:, 0:H], straw[:, H:2 * H],
                             bng_ref[...], bnb_ref[...])
    y1n = y1_ref[...] * sc + sh
    u = jnp.dot(y1n, w2_ref[...], preferred_element_type=_f32) + b2_ref[...]
    y2 = jnp.maximum(u, u * SLOPE)
    y2_ref[...] = y2

    @pl.when(i == 0)
    def _():
        st2_ref[...] = jnp.zeros_like(st2_ref)

    st2_ref[0:1, :] += jnp.sum(y2, axis=0, keepdims=True)
    st2_ref[1:2, :] += jnp.sum(y2 * y2, axis=0, keepdims=True)


def _tc_p2(st1, bng, bnb, w2, b2, y1):
    full = lambda shp: pl.BlockSpec(shp, lambda i: (0, 0))
    return pl.pallas_call(
        _tc_p2_body,
        grid=(NBLKE,),
        in_specs=[full((NW, 2 * H)), full((1, H)), full((1, H)),
                  full((H, H)), full((1, H)),
                  pl.BlockSpec((RBE, H), lambda i: (i, 0))],
        out_specs=[pl.BlockSpec((RBE, H), lambda i: (i, 0)),
                   pl.BlockSpec((8, H), lambda i: (0, 0))],
        out_shape=(jax.ShapeDtypeStruct((E, H), _f32),
                   jax.ShapeDtypeStruct((8, H), _f32)),
    )(st1, bng, bnb, w2, b2, y1)


def _tc_p3a_body(st2_ref, bng_ref, bnb_ref, w3_ref, b3_ref, y2_ref, coef_ref,
                 cond_ref):
    sc, sh = _bn_scale_shift(st2_ref[0:1, :], st2_ref[1:2, :],
                             bng_ref[...], bnb_ref[...])
    y2n = y2_ref[...] * sc + sh
    fe = jnp.dot(y2n, w3_ref[...], preferred_element_type=_f32) + b3_ref[...]
    cond_ref[...] = fe * coef_ref[...]


def _tc_p3a(st2, bng, bnb, w3, b3, y2, coef):
    full = lambda shp: pl.BlockSpec(shp, lambda i: (0, 0))
    return pl.pallas_call(
        _tc_p3a_body,
        grid=(NBLKE,),
        in_specs=[full((8, H)), full((1, H)), full((1, H)),
                  full((H, H)), full((1, H)),
                  pl.BlockSpec((RBE, H), lambda i: (i, 0)),
                  pl.BlockSpec((RBE, 1), lambda i: (i, 0))],
        out_specs=pl.BlockSpec((RBE, H), lambda i: (i, 0)),
        out_shape=jax.ShapeDtypeStruct((E, H), _f32),
    )(st2, bng, bnb, w3, b3, y2, coef)


def _node_update(p0_ref, p1_ref, d_ref, h_ref, v_ref, w0_ref, b0_ref,
                 g_ref, bb_ref, w2_ref, b2_ref):
    z = (v_ref[...] * d_ref[...] + p0_ref[0, :N, :] + p1_ref[0, :N, :])
    z1 = jnp.dot(z, w0_ref[...], preferred_element_type=_f32) + b0_ref[...]
    z1 = _leaky(z1)
    m = jnp.mean(z1, axis=0, keepdims=True)
    cz = z1 - m
    var = jnp.mean(cz * cz, axis=0, keepdims=True)
    z1n = cz * lax.rsqrt(var + EPS) * g_ref[...] + bb_ref[...]
    z2 = jnp.dot(z1n, w2_ref[...], preferred_element_type=_f32) + b2_ref[...]
    return z2 + h_ref[...]


def _tc_node_body(p0_ref, p1_ref, d_ref, h_ref, v_ref, w0_ref, b0_ref,
                  g_ref, bb_ref, w2_ref, b2_ref, ho_ref):
    ho_ref[...] = _node_update(p0_ref, p1_ref, d_ref, h_ref, v_ref, w0_ref,
                               b0_ref, g_ref, bb_ref, w2_ref, b2_ref)


def _tc_nodef_body(p0_ref, p1_ref, d_ref, h_ref, v_ref, w0_ref, b0_ref,
                   g_ref, bb_ref, w2_ref, b2_ref,
                   wa_ref, wc_ref, wd_ref, bd_ref,
                   ho_ref, a_ref, c_ref, dn_ref):
    hn = _node_update(p0_ref, p1_ref, d_ref, h_ref, v_ref, w0_ref,
                      b0_ref, g_ref, bb_ref, w2_ref, b2_ref)
    ho_ref[...] = hn
    a_ref[...] = jnp.dot(hn, wa_ref[...], preferred_element_type=_f32)
    c_ref[...] = jnp.dot(hn, wc_ref[...], preferred_element_type=_f32)
    dn_ref[...] = jnp.dot(hn, wd_ref[...], preferred_element_type=_f32) + bd_ref[...]


def _fs(shp):
    return pl.BlockSpec(shp, lambda i: tuple(0 for _ in shp))


def _parts_specs():
    return [pl.BlockSpec((1, NPAD, H), lambda i: (0, 0, 0)),
            pl.BlockSpec((1, NPAD, H), lambda i: (1, 0, 0))]


_NODE_TAIL = [(N, H), (N, H), (1, H), (H, H), (1, H), (1, H), (1, H),
              (H, H), (1, H)]
_PROJ_TAIL = [(H, H), (H, H), (H, H), (1, H)]


def _tc_node(parts, D, h, v, w0, b0, g, bb, w2, b2):
    return pl.pallas_call(
        _tc_node_body,
        grid=(1,),
        in_specs=_parts_specs() + [_fs(s) for s in _NODE_TAIL],
        out_specs=_fs((N, H)),
        out_shape=jax.ShapeDtypeStruct((N, H), _f32),
    )(parts, parts, D, h, v, w0, b0, g, bb, w2, b2)


def _tc_nodef(parts, D, h, v, w0, b0, g, bb, w2, b2, wa, wc, wd, bd):
    return pl.pallas_call(
        _tc_nodef_body,
        grid=(1,),
        in_specs=_parts_specs() + [_fs(s) for s in _NODE_TAIL + _PROJ_TAIL],
        out_specs=[_fs((N, H))] * 4,
        out_shape=(jax.ShapeDtypeStruct((N, H), _f32),) * 4,
    )(parts, parts, D, h, v, w0, b0, g, bb, w2, b2, wa, wc, wd, bd)


def _tc_head_body(h_ref, w0_ref, b0_ref, g_ref, bb_ref, w1_ref, b1_ref, y_ref,
                  loss_ref, pred_ref):
    t = jnp.dot(h_ref[...], w0_ref[...], preferred_element_type=_f32) + b0_ref[...]
    t = _leaky(t)
    m = jnp.mean(t, axis=0, keepdims=True)
    ct = t - m
    var = jnp.mean(ct * ct, axis=0, keepdims=True)
    tn = ct * lax.rsqrt(var + EPS) * g_ref[...] + bb_ref[...]
    pred = jnp.dot(tn, w1_ref[...], preferred_element_type=_f32) + b1_ref[...]
    pred_ref[...] = pred
    r = pred - y_ref[...]
    loss_ref[...] = jnp.mean(r * r).reshape(1, 1)


def _tc_head(h, w0, b0, g, bb, w1, b1, y):
    return pl.pallas_call(
        _tc_head_body,
        out_shape=(jax.ShapeDtypeStruct((1, 1), _f32),
                   jax.ShapeDtypeStruct((N, 1), _f32)),
    )(h, w0, b0, g, bb, w1, b1, y)


def _row(v):
    return v.reshape(1, -1).astype(_f32)


def kernel(x, edge_attr, edge_index, y, params):
    x = x.astype(_f32)
    edge_attr = edge_attr.astype(_f32)
    y = y.astype(_f32)
    src = edge_index[0].astype(jnp.int32)
    dst = edge_index[1].astype(jnp.int32)

    p = params
    wfb = p["f_b"]["W"].astype(_f32)
    wfb_pad = jnp.zeros((7, CP, H), _f32).at[:, :STEPS, :].set(
        wfb.reshape(7, STEPS, H)).reshape(7 * CP, H)
    w1m_all = jnp.concatenate(
        [p["layers"][l]["f_e"][0]["W"][H:2 * H, :] for l in range(3)], axis=0)
    b1_all = jnp.stack([p["layers"][l]["f_e"][0]["b"] for l in range(3)], axis=0)

    ea4d = edge_attr[:, 3].reshape(NBLK, RB // H, H)
    g0, g1, g2, coef2 = _tc_pre(edge_attr, ea4d, wfb_pad, _row(p["f_b"]["b"]),
                                w1m_all.astype(_f32), b1_all.astype(_f32))
    coef = coef2.reshape(E, 1)
    gmids = (g0, g1, g2)

    def proj_w(l):
        lp = p["layers"][l]
        w1 = lp["f_e"][0]["W"].astype(_f32)
        return (w1[:H, :], w1[2 * H:, :],
                lp["f_d"]["W"].astype(_f32), _row(lp["f_d"]["b"]))

    h, A, C, D = _tc_init(x, p["f_x"]["W"].astype(_f32), _row(p["f_x"]["b"]),
                          *proj_w(0))

    for l in range(3):
        lp = p["layers"][l]
        y1, st1p = _sc_p1(A, C, gmids[l], src, dst)
        y2, st2 = _tc_p2(st1p.reshape(NW, 2 * H),
                         _row(lp["f_e"][1]["g"]), _row(lp["f_e"][1]["b"]),
                         lp["f_e"][2]["W"].astype(_f32), _row(lp["f_e"][2]["b"]),
                         y1)
        cond = _tc_p3a(st2, _row(lp["f_e"][3]["g"]), _row(lp["f_e"][3]["b"]),
                       lp["f_e"][4]["W"].astype(_f32), _row(lp["f_e"][4]["b"]),
                       y2, coef)
        parts = _sc_p3(cond, D, src, dst).reshape(2, NPAD, H)
        node_args = (parts, D, h, lp["v"].astype(_f32),
                     lp["f_n"][0]["W"].astype(_f32), _row(lp["f_n"][0]["b"]),
                     _row(lp["f_n"][1]["g"]), _row(lp["f_n"][1]["b"]),
                     lp["f_n"][2]["W"].astype(_f32), _row(lp["f_n"][2]["b"]))
        if l < 2:
            h, A, C, D = _tc_nodef(*node_args, *proj_w(l + 1))
        else:
            h = _tc_node(*node_args)

    ft = p["f_target"]
    loss, pred = _tc_head(h, ft[0]["W"].astype(_f32), _row(ft[0]["b"]),
                          _row(ft[1]["g"]), _row(ft[1]["b"]),
                          ft[2]["W"].astype(_f32), _row(ft[2]["b"]), y)
    return loss[0, 0], pred


# speedup vs baseline: 1.3823x; 1.0046x over previous
import functools

import jax
import jax.numpy as jnp
from jax import lax
from jax.experimental import pallas as pl
from jax.experimental.pallas import tpu as pltpu
from jax.experimental.pallas import tpu_sc as plsc

N = 10000
E = 160000
H = 128
STEPS = 50
CP = 64
EPS = 1e-5
SLOPE = 0.01
E_F = float(E)

RB = 6400
NBLK = E // RB
RBE = 16000
NBLKE = E // RBE
K = 128
NCHUNK = E // K
K3 = 80
NCHUNK3 = E // K3
NC, NS = 2, 16
NW = NC * NS
NPAD = 10240
ROWS_PER_TILE = NPAD // NS

_f32 = jnp.float32


def _leaky(x):
    return jnp.maximum(x, x * SLOPE)


def _sc_p1_body(a_hbm, c_hbm, g_hbm, src_hbm, dst_hbm, y_hbm, st_hbm,
                idx_s0, idx_d0, a_v0, c_v0, g_v0,
                idx_s1, idx_d1, a_v1, c_v1, g_v1,
                st_v, semg0, semg1, semy0, semy1, semi0, semi1):
    cid = lax.axis_index("c")
    sid = lax.axis_index("s")
    w = cid * NS + sid
    nchunks = (NCHUNK - w + NW - 1) // NW
    bufs = ((idx_s0, idx_d0, a_v0, c_v0, g_v0, semg0, semy0, semi0),
            (idx_s1, idx_d1, a_v1, c_v1, g_v1, semg1, semy1, semi1))
    zero = jnp.zeros((16,), _f32)
    init = (zero,) * 16
    for k in range(16):
        st_v[pl.ds(k * 16, 16)] = zero

    def cbase(k):
        return (w + k * NW) * K

    def fire_idx(k, buf):
        idx_s, idx_d = buf[0], buf[1]
        semi = buf[7]
        base = cbase(k)
        pltpu.async_copy(src_hbm.at[pl.ds(base, K)], idx_s, semi)
        pltpu.async_copy(dst_hbm.at[pl.ds(base, K)], idx_d, semi)

    def wait_idx(k, buf):
        idx_s, idx_d = buf[0], buf[1]
        semi = buf[7]
        base = cbase(k)
        pltpu.make_async_copy(src_hbm.at[pl.ds(base, K)], idx_s, semi).wait()
        pltpu.make_async_copy(dst_hbm.at[pl.ds(base, K)], idx_d, semi).wait()

    def fire_gather(k, buf):
        idx_s, idx_d, a_v, c_v, g_v, semg = buf[:6]
        base = cbase(k)
        pltpu.async_copy(a_hbm.at[idx_s], a_v, semg)
        pltpu.async_copy(c_hbm.at[idx_d], c_v, semg)
        pltpu.async_copy(g_hbm.at[pl.ds(base, K)], g_v, semg)

    def process(k, b):
        idx_s, idx_d, a_v, c_v, g_v, semg, semy, _ = bufs[b]
        a_vo = bufs[1 - b][2]
        semyo = bufs[1 - b][6]

        @pl.when(k >= 1)
        def _():
            pltpu.make_async_copy(y_hbm.at[pl.ds(0, K)], a_vo, semyo).wait()

        @pl.when(k + 1 < nchunks)
        def _():
            wait_idx(k + 1, bufs[1 - b])
            fire_gather(k + 1, bufs[1 - b])

        base = cbase(k)
        pltpu.make_async_copy(a_hbm.at[idx_s], a_v, semg).wait()
        pltpu.make_async_copy(c_hbm.at[idx_d], c_v, semg).wait()
        pltpu.make_async_copy(g_hbm.at[pl.ds(base, K)], g_v, semg).wait()

        @pl.when(k + 2 < nchunks)
        def _():
            fire_idx(k + 2, bufs[b])

        def row_body(i, st_in):
            out = list(st_in)
            for kk in range(8):
                sl = pl.ds(kk * 16, 16)
                p = a_v[i, sl] + c_v[i, sl] + g_v[i, sl]
                yv = jnp.maximum(p, p * SLOPE)
                a_v[i, sl] = yv
                out[kk] = out[kk] + yv
                out[8 + kk] = out[8 + kk] + yv * yv
            return tuple(out)

        st = lax.fori_loop(0, K, row_body, init)
        for kk in range(16):
            sl = pl.ds(kk * 16, 16)
            st_v[sl] = st_v[sl] + st[kk]
        pltpu.async_copy(a_v, y_hbm.at[pl.ds(base, K)], semy)

    fire_idx(0, bufs[0])
    wait_idx(0, bufs[0])
    fire_gather(0, bufs[0])

    @pl.when(1 < nchunks)
    def _():
        fire_idx(1, bufs[1])

    def pair_body(p, carry):
        process(2 * p, 0)

        @pl.when(2 * p + 1 < nchunks)
        def _():
            process(2 * p + 1, 1)

        return carry

    lax.fori_loop(0, (nchunks + 1) // 2, pair_body, 0)
    last = (nchunks - 1) % 2

    @pl.when(last == 0)
    def _():
        pltpu.make_async_copy(y_hbm.at[pl.ds(0, K)], a_v0, semy0).wait()

    @pl.when(last == 1)
    def _():
        pltpu.make_async_copy(y_hbm.at[pl.ds(0, K)], a_v1, semy1).wait()

    pltpu.sync_copy(st_v, st_hbm.at[pl.ds(w * 2 * H, 2 * H)])


def _sc_p1(A, C, G, src, dst):
    mesh = plsc.VectorSubcoreMesh(core_axis_name="c", subcore_axis_name="s")
    buf_set = [
        pltpu.VMEM((K,), jnp.int32),
        pltpu.VMEM((K,), jnp.int32),
        pltpu.VMEM((K, H), _f32),
        pltpu.VMEM((K, H), _f32),
        pltpu.VMEM((K, H), _f32),
    ]
    f = pl.kernel(
        _sc_p1_body,
        out_type=(jax.ShapeDtypeStruct((E, H), _f32),
                  jax.ShapeDtypeStruct((NW * 2 * H,), _f32)),
        mesh=mesh,
        scratch_types=buf_set + buf_set + [
            pltpu.VMEM((2 * H,), _f32),
            pltpu.SemaphoreType.DMA,
            pltpu.SemaphoreType.DMA,
            pltpu.SemaphoreType.DMA,
            pltpu.SemaphoreType.DMA,
            pltpu.SemaphoreType.DMA,
            pltpu.SemaphoreType.DMA,
        ],
    )
    return f(A, C, G, src, dst)


def _sc_p3_body(cond_hbm, d_hbm, src_hbm, dst_hbm, out_hbm,
                idx_s0, idx_d0, m_v0, d_v0,
                idx_s1, idx_d1, m_v1, d_v1,
                acc, semg0, semg1, semsc0, semsc1, semi0, semi1):
    cid = lax.axis_index("c")
    sid = lax.axis_index("s")
    w = cid * NS + sid
    bufs = ((idx_s0, idx_d0, m_v0, d_v0, semg0, semsc0, semi0),
            (idx_s1, idx_d1, m_v1, d_v1, semg1, semsc1, semi1))

    def zrow(i, _):
        for k in range(8):
            m_v0[i, pl.ds(k * 16, 16)] = jnp.zeros((16,), _f32)
        return 0

    lax.fori_loop(0, K3, zrow, 0)
    base_r = sid * ROWS_PER_TILE
    for t in range(ROWS_PER_TILE // K3):
        pltpu.sync_copy(m_v0, acc.at[pl.ds(base_r + t * K3, K3)])
    plsc.subcore_barrier()

    nchunks = (NCHUNK3 - w + NW - 1) // NW

    def cbase(k):
        return (w + k * NW) * K3

    def fire_idx_s(k, buf):
        base = cbase(k)
        pltpu.async_copy(src_hbm.at[pl.ds(base, K3)], buf[0], buf[6])

    def wait_idx_s(k, buf):
        base = cbase(k)
        pltpu.make_async_copy(src_hbm.at[pl.ds(base, K3)], buf[0],
                              buf[6]).wait()

    def fire_gather(k, buf):
        idx_s, idx_d, m_v, d_v, semg = buf[:5]
        base = cbase(k)
        pltpu.async_copy(d_hbm.at[idx_s], d_v, semg)
        pltpu.async_copy(cond_hbm.at[pl.ds(base, K3)], m_v, semg)
        pltpu.async_copy(dst_hbm.at[pl.ds(base, K3)], idx_d, semg)

    def process(k, b):
        idx_s, idx_d, m_v, d_v, semg, semsc, _ = bufs[b]
        idx_do, m_vo = bufs[1 - b][1], bufs[1 - b][2]
        semsco = bufs[1 - b][5]

        @pl.when(k >= 1)
        def _():
            pltpu.make_async_copy(m_vo, acc.at[idx_do], semsco).wait()

        @pl.when(k + 1 < nchunks)
        def _():
            wait_idx_s(k + 1, bufs[1 - b])
            fire_gather(k + 1, bufs[1 - b])

        base = cbase(k)
        pltpu.make_async_copy(d_hbm.at[idx_s], d_v, semg).wait()
        pltpu.make_async_copy(cond_hbm.at[pl.ds(base, K3)], m_v, semg).wait()
        pltpu.make_async_copy(dst_hbm.at[pl.ds(base, K3)], idx_d, semg).wait()

        @pl.when(k + 2 < nchunks)
        def _():
            fire_idx_s(k + 2, bufs[b])

        def row_body(i, _):
            for kk in range(8):
                sl = pl.ds(kk * 16, 16)
                m_v[i, sl] = m_v[i, sl] * d_v[i, sl]
            return 0

        lax.fori_loop(0, K3, row_body, 0)
        pltpu.async_copy(m_v, acc.at[idx_d], semsc, add=True)

    fire_idx_s(0, bufs[0])
    wait_idx_s(0, bufs[0])
    fire_gather(0, bufs[0])

    @pl.when(1 < nchunks)
    def _():
        fire_idx_s(1, bufs[1])

    def pair_body(p, carry):
        process(2 * p, 0)

        @pl.when(2 * p + 1 < nchunks)
        def _():
            process(2 * p + 1, 1)

        return carry

    lax.fori_loop(0, (nchunks + 1) // 2, pair_body, 0)
    last = (nchunks - 1) % 2

    @pl.when(last == 0)
    def _():
        pltpu.make_async_copy(m_v0, acc.at[idx_d0], semsc0).wait()

    @pl.when(last == 1)
    def _():
        pltpu.make_async_copy(m_v1, acc.at[idx_d1], semsc1).wait()

    plsc.subcore_barrier()

    out_base = cid * NPAD + base_r
    for t in range(ROWS_PER_TILE // K3):
        pltpu.sync_copy(acc.at[pl.ds(base_r + t * K3, K3)], m_v0)
        pltpu.sync_copy(m_v0, out_hbm.at[pl.ds(out_base + t * K3, K3)])


def _sc_p3(cond, D, src, dst):
    mesh = plsc.VectorSubcoreMesh(core_axis_name="c", subcore_axis_name="s")
    buf_set = [
        pltpu.VMEM((K3,), jnp.int32),
        pltpu.VMEM((K3,), jnp.int32),
        pltpu.VMEM((K3, H), _f32),
        pltpu.VMEM((K3, H), _f32),
    ]
    f = pl.kernel(
        _sc_p3_body,
        out_type=jax.ShapeDtypeStruct((NC * NPAD, H), _f32),
        mesh=mesh,
        scratch_types=buf_set + buf_set + [
            pltpu.VMEM_SHARED((NPAD, H), _f32),
            pltpu.SemaphoreType.DMA,
            pltpu.SemaphoreType.DMA,
            pltpu.SemaphoreType.DMA,
            pltpu.SemaphoreType.DMA,
            pltpu.SemaphoreType.DMA,
            pltpu.SemaphoreType.DMA,
        ],
    )
    return f(cond, D, src, dst)


def _tc_pre_body(ea_ref, ea4_ref, wfbp_ref, bfb_ref, w1m_ref, b1_ref,
                 g0_ref, g1_ref, g2_ref, coef_ref):
    ea = ea_ref[...]
    centers = lax.broadcasted_iota(jnp.int32, (1, CP), 1).astype(_f32) / (STEPS - 1.0)
    g = jnp.zeros((RB, H), _f32) + bfb_ref[...]
    for a in range(7):
        col = ea[:, a:a + 1]
        dexp = jnp.exp(-((col - centers) ** 2) * STEPS)
        g = g + jnp.dot(dexp, wfbp_ref[pl.ds(a * CP, CP), :],
                        preferred_element_type=_f32)
    for l, gref in enumerate((g0_ref, g1_ref, g2_ref)):
        gref[...] = jnp.dot(g, w1m_ref[pl.ds(l * H, H), :],
                            preferred_element_type=_f32) + b1_ref[l:l + 1, :]
    coef_ref[...] = jnp.cos(1.5707963267948966 * ea4_ref[...])


def _tc_pre(edge_attr, ea4d, wfb_pad, bfb, w1m_all, b1_all):
    out_shape = (jax.ShapeDtypeStruct((E, H), _f32),) * 3 + (
        jax.ShapeDtypeStruct((NBLK, RB // H, H), _f32),)
    full = lambda shp: pl.BlockSpec(shp, lambda i: (0, 0))
    return pl.pallas_call(
        _tc_pre_body,
        grid=(NBLK,),
        in_specs=[pl.BlockSpec((RB, 7), lambda i: (i, 0)),
                  pl.BlockSpec((1, RB // H, H), lambda i: (i, 0, 0)),
                  full((7 * CP, H)), full((1, H)),
                  full((3 * H, H)), full((3, H))],
        out_specs=[pl.BlockSpec((RB, H), lambda i: (i, 0))] * 3 +
                  [pl.BlockSpec((1, RB // H, H), lambda i: (i, 0, 0))],
        out_shape=out_shape,
    )(edge_attr, ea4d, wfb_pad, bfb, w1m_all, b1_all)


def _tc_init_body(x_ref, w_ref, b_ref, wa_ref, wc_ref, wd_ref, bd_ref,
                  h_ref, a_ref, c_ref, d_ref):
    h = jnp.dot(x_ref[...], w_ref[...], preferred_element_type=_f32) + b_ref[...]
    h_ref[...] = h
    a_ref[...] = jnp.dot(h, wa_ref[...], preferred_element_type=_f32)
    c_ref[...] = jnp.dot(h, wc_ref[...], preferred_element_type=_f32)
    d_ref[...] = jnp.dot(h, wd_ref[...], preferred_element_type=_f32) + bd_ref[...]


def _tc_init(x, w, b, wa, wc, wd, bd):
    return pl.pallas_call(
        _tc_init_body,
        out_shape=(jax.ShapeDtypeStruct((N, H), _f32),) * 4,
    )(x, w, b, wa, wc, wd, bd)


def _tc_nodepre_body(h_ref, wa_ref, wc_ref, wd_ref, bd_ref, a_ref, c_ref, d_ref):
    h = h_ref[...]
    a_ref[...] = jnp.dot(h, wa_ref[...], preferred_element_type=_f32)
    c_ref[...] = jnp.dot(h, wc_ref[...], preferred_element_type=_f32)
    d_ref[...] = jnp.dot(h, wd_ref[...], preferred_element_type=_f32) + bd_ref[...]


def _tc_nodepre(h, wa, wc, wd, bd):
    return pl.pallas_call(
        _tc_nodepre_body,
        out_shape=(jax.ShapeDtypeStruct((N, H), _f32),) * 3,
    )(h, wa, wc, wd, bd)


def _bn_scale_shift(st_row_sum, st_row_sq, g, b):
    m = st_row_sum / E_F
    var = st_row_sq / E_F - m * m
    sc = g * lax.rsqrt(var + EPS)
    sh = b - m * sc
    return sc, sh


def _tc_p2_body(st_ref, bng_ref, bnb_ref, w2_ref, b2_ref, y1_ref,
                y2_ref, st2_ref):
    i = pl.program_id(0)
    straw = jnp.sum(st_ref[...], axis=0, keepdims=True)
    sc, sh = _bn_scale_shift(straw[:, 0:H], straw[:, H:2 * H],
                             bng_ref[...], bnb_ref[...])
    y1n = y1_ref[...] * sc + sh
    u = jnp.dot(y1n, w2_ref[...], preferred_element_type=_f32) + b2_ref[...]
    y2 = jnp.maximum(u, u * SLOPE)
    y2_ref[...] = y2

    @pl.when(i == 0)
    def _():
        st2_ref[...] = jnp.zeros_like(st2_ref)

    st2_ref[0:1, :] += jnp.sum(y2, axis=0, keepdims=True)
    st2_ref[1:2, :] += jnp.sum(y2 * y2, axis=0, keepdims=True)


def _tc_p2(st1, bng, bnb, w2, b2, y1):
    full = lambda shp: pl.BlockSpec(shp, lambda i: (0, 0))
    return pl.pallas_call(
        _tc_p2_body,
        grid=(NBLKE,),
        in_specs=[full((NW, 2 * H)), full((1, H)), full((1, H)),
                  full((H, H)), full((1, H)),
                  pl.BlockSpec((RBE, H), lambda i: (i, 0))],
        out_specs=[pl.BlockSpec((RBE, H), lambda i: (i, 0)),
                   pl.BlockSpec((8, H), lambda i: (0, 0))],
        out_shape=(jax.ShapeDtypeStruct((E, H), _f32),
                   jax.ShapeDtypeStruct((8, H), _f32)),
    )(st1, bng, bnb, w2, b2, y1)


def _tc_p3a_body(st2_ref, bng_ref, bnb_ref, w3_ref, b3_ref, y2_ref, coef_ref,
                 cond_ref):
    sc, sh = _bn_scale_shift(st2_ref[0:1, :], st2_ref[1:2, :],
                             bng_ref[...], bnb_ref[...])
    y2n = y2_ref[...] * sc + sh
    fe = jnp.dot(y2n, w3_ref[...], preferred_element_type=_f32) + b3_ref[...]
    cond_ref[...] = fe * coef_ref[...]


def _tc_p3a(st2, bng, bnb, w3, b3, y2, coef):
    full = lambda shp: pl.BlockSpec(shp, lambda i: (0, 0))
    return pl.pallas_call(
        _tc_p3a_body,
        grid=(NBLKE,),
        in_specs=[full((8, H)), full((1, H)), full((1, H)),
                  full((H, H)), full((1, H)),
                  pl.BlockSpec((RBE, H), lambda i: (i, 0)),
                  pl.BlockSpec((RBE, 1), lambda i: (i, 0))],
        out_specs=pl.BlockSpec((RBE, H), lambda i: (i, 0)),
        out_shape=jax.ShapeDtypeStruct((E, H), _f32),
    )(st2, bng, bnb, w3, b3, y2, coef)


def _node_update(p0_ref, p1_ref, d_ref, h_ref, v_ref, w0_ref, b0_ref,
                 g_ref, bb_ref, w2_ref, b2_ref):
    z = (v_ref[...] * d_ref[...] + p0_ref[0, :N, :] + p1_ref[0, :N, :])
    z1 = jnp.dot(z, w0_ref[...], preferred_element_type=_f32) + b0_ref[...]
    z1 = _leaky(z1)
    m = jnp.mean(z1, axis=0, keepdims=True)
    cz = z1 - m
    var = jnp.mean(cz * cz, axis=0, keepdims=True)
    z1n = cz * lax.rsqrt(var + EPS) * g_ref[...] + bb_ref[...]
    z2 = jnp.dot(z1n, w2_ref[...], preferred_element_type=_f32) + b2_ref[...]
    return z2 + h_ref[...]


def _tc_node_body(p0_ref, p1_ref, d_ref, h_ref, v_ref, w0_ref, b0_ref,
                  g_ref, bb_ref, w2_ref, b2_ref, ho_ref):
    ho_ref[...] = _node_update(p0_ref, p1_ref, d_ref, h_ref, v_ref, w0_ref,
                               b0_ref, g_ref, bb_ref, w2_ref, b2_ref)


def _tc_nodef_body(p0_ref, p1_ref, d_ref, h_ref, v_ref, w0_ref, b0_ref,
                   g_ref, bb_ref, w2_ref, b2_ref,
                   wa_ref, wc_ref, wd_ref, bd_ref,
                   ho_ref, a_ref, c_ref, dn_ref):
    hn = _node_update(p0_ref, p1_ref, d_ref, h_ref, v_ref, w0_ref,
                      b0_ref, g_ref, bb_ref, w2_ref, b2_ref)
    ho_ref[...] = hn
    a_ref[...] = jnp.dot(hn, wa_ref[...], preferred_element_type=_f32)
    c_ref[...] = jnp.dot(hn, wc_ref[...], preferred_element_type=_f32)
    dn_ref[...] = jnp.dot(hn, wd_ref[...], preferred_element_type=_f32) + bd_ref[...]


def _fs(shp):
    return pl.BlockSpec(shp, lambda i: tuple(0 for _ in shp))


def _parts_specs():
    return [pl.BlockSpec((1, NPAD, H), lambda i: (0, 0, 0)),
            pl.BlockSpec((1, NPAD, H), lambda i: (1, 0, 0))]


_NODE_TAIL = [(N, H), (N, H), (1, H), (H, H), (1, H), (1, H), (1, H),
              (H, H), (1, H)]
_PROJ_TAIL = [(H, H), (H, H), (H, H), (1, H)]


def _tc_nodeh_body(p0_ref, p1_ref, d_ref, h_ref, v_ref, w0_ref, b0_ref,
                   g_ref, bb_ref, w2_ref, b2_ref,
                   tw0_ref, tb0_ref, tg_ref, tbb_ref, tw1_ref, tb1_ref, y_ref,
                   loss_ref, pred_ref):
    hn = _node_update(p0_ref, p1_ref, d_ref, h_ref, v_ref, w0_ref,
                      b0_ref, g_ref, bb_ref, w2_ref, b2_ref)
    t = jnp.dot(hn, tw0_ref[...], preferred_element_type=_f32) + tb0_ref[...]
    t = _leaky(t)
    m = jnp.mean(t, axis=0, keepdims=True)
    ct = t - m
    var = jnp.mean(ct * ct, axis=0, keepdims=True)
    tn = ct * lax.rsqrt(var + EPS) * tg_ref[...] + tbb_ref[...]
    pred = jnp.dot(tn, tw1_ref[...], preferred_element_type=_f32) + tb1_ref[...]
    pred_ref[...] = pred
    r = pred - y_ref[...]
    loss_ref[...] = jnp.mean(r * r).reshape(1, 1)


_HEAD_TAIL = [(H, H // 2), (1, H // 2), (1, H // 2), (1, H // 2),
              (H // 2, 1), (1, 1), (N, 1)]


def _tc_nodeh(parts, D, h, v, w0, b0, g, bb, w2, b2,
              tw0, tb0, tg, tbb, tw1, tb1, y):
    return pl.pallas_call(
        _tc_nodeh_body,
        grid=(1,),
        in_specs=_parts_specs() + [_fs(s) for s in _NODE_TAIL + _HEAD_TAIL],
        out_specs=[_fs((1, 1)), _fs((N, 1))],
        out_shape=(jax.ShapeDtypeStruct((1, 1), _f32),
                   jax.ShapeDtypeStruct((N, 1), _f32)),
    )(parts, parts, D, h, v, w0, b0, g, bb, w2, b2,
      tw0, tb0, tg, tbb, tw1, tb1, y)


def _tc_nodef(parts, D, h, v, w0, b0, g, bb, w2, b2, wa, wc, wd, bd):
    return pl.pallas_call(
        _tc_nodef_body,
        grid=(1,),
        in_specs=_parts_specs() + [_fs(s) for s in _NODE_TAIL + _PROJ_TAIL],
        out_specs=[_fs((N, H))] * 4,
        out_shape=(jax.ShapeDtypeStruct((N, H), _f32),) * 4,
    )(parts, parts, D, h, v, w0, b0, g, bb, w2, b2, wa, wc, wd, bd)


def _tc_head_body(h_ref, w0_ref, b0_ref, g_ref, bb_ref, w1_ref, b1_ref, y_ref,
                  loss_ref, pred_ref):
    t = jnp.dot(h_ref[...], w0_ref[...], preferred_element_type=_f32) + b0_ref[...]
    t = _leaky(t)
    m = jnp.mean(t, axis=0, keepdims=True)
    ct = t - m
    var = jnp.mean(ct * ct, axis=0, keepdims=True)
    tn = ct * lax.rsqrt(var + EPS) * g_ref[...] + bb_ref[...]
    pred = jnp.dot(tn, w1_ref[...], preferred_element_type=_f32) + b1_ref[...]
    pred_ref[...] = pred
    r = pred - y_ref[...]
    loss_ref[...] = jnp.mean(r * r).reshape(1, 1)


def _tc_head(h, w0, b0, g, bb, w1, b1, y):
    return pl.pallas_call(
        _tc_head_body,
        out_shape=(jax.ShapeDtypeStruct((1, 1), _f32),
                   jax.ShapeDtypeStruct((N, 1), _f32)),
    )(h, w0, b0, g, bb, w1, b1, y)


def _row(v):
    return v.reshape(1, -1).astype(_f32)


def kernel(x, edge_attr, edge_index, y, params):
    x = x.astype(_f32)
    edge_attr = edge_attr.astype(_f32)
    y = y.astype(_f32)
    src = edge_index[0].astype(jnp.int32)
    dst = edge_index[1].astype(jnp.int32)

    p = params
    wfb = p["f_b"]["W"].astype(_f32)
    wfb_pad = jnp.zeros((7, CP, H), _f32).at[:, :STEPS, :].set(
        wfb.reshape(7, STEPS, H)).reshape(7 * CP, H)
    w1m_all = jnp.concatenate(
        [p["layers"][l]["f_e"][0]["W"][H:2 * H, :] for l in range(3)], axis=0)
    b1_all = jnp.stack([p["layers"][l]["f_e"][0]["b"] for l in range(3)], axis=0)

    ea4d = edge_attr[:, 3].reshape(NBLK, RB // H, H)
    g0, g1, g2, coef2 = _tc_pre(edge_attr, ea4d, wfb_pad, _row(p["f_b"]["b"]),
                                w1m_all.astype(_f32), b1_all.astype(_f32))
    coef = coef2.reshape(E, 1)
    gmids = (g0, g1, g2)

    def proj_w(l):
        lp = p["layers"][l]
        w1 = lp["f_e"][0]["W"].astype(_f32)
        return (w1[:H, :], w1[2 * H:, :],
                lp["f_d"]["W"].astype(_f32), _row(lp["f_d"]["b"]))

    h, A, C, D = _tc_init(x, p["f_x"]["W"].astype(_f32), _row(p["f_x"]["b"]),
                          *proj_w(0))

    for l in range(3):
        lp = p["layers"][l]
        y1, st1p = _sc_p1(A, C, gmids[l], src, dst)
        y2, st2 = _tc_p2(st1p.reshape(NW, 2 * H),
                         _row(lp["f_e"][1]["g"]), _row(lp["f_e"][1]["b"]),
                         lp["f_e"][2]["W"].astype(_f32), _row(lp["f_e"][2]["b"]),
                         y1)
        cond = _tc_p3a(st2, _row(lp["f_e"][3]["g"]), _row(lp["f_e"][3]["b"]),
                       lp["f_e"][4]["W"].astype(_f32), _row(lp["f_e"][4]["b"]),
                       y2, coef)
        parts = _sc_p3(cond, D, src, dst).reshape(2, NPAD, H)
        node_args = (parts, D, h, lp["v"].astype(_f32),
                     lp["f_n"][0]["W"].astype(_f32), _row(lp["f_n"][0]["b"]),
                     _row(lp["f_n"][1]["g"]), _row(lp["f_n"][1]["b"]),
                     lp["f_n"][2]["W"].astype(_f32), _row(lp["f_n"][2]["b"]))
        if l < 2:
            h, A, C, D = _tc_nodef(*node_args, *proj_w(l + 1))
        else:
            ft = p["f_target"]
            loss, pred = _tc_nodeh(*node_args,
                                   ft[0]["W"].astype(_f32), _row(ft[0]["b"]),
                                   _row(ft[1]["g"]), _row(ft[1]["b"]),
                                   ft[2]["W"].astype(_f32), _row(ft[2]["b"]),
                                   y)
    return loss[0, 0], pred
